# Initial kernel scaffold; baseline (speedup 1.0000x reference)
import jax
import jax.numpy as jnp
from jax.experimental import pallas as pl


def kernel(x, edge_index, t, noise, W_in, b_in, gat_W, gat_al, gat_ar, W_out, b_out, time_emb):
    raise NotImplementedError("write your pallas kernel here")


# speedup vs baseline: 1.0389x; 1.0389x over previous
import functools
import jax
import jax.numpy as jnp
from jax import lax
from jax.experimental import pallas as pl
from jax.experimental.pallas import tpu as pltpu

N = 10000
D = 128
E = 320000
H = 256
L = 2
NHEAD = 4
HD = 64
T = 1000

NP = 10240
TP = 1024
BN = 1024
GN = NP // BN


def _rln(v, eps=1e-5):
    m = v.mean(axis=-1, keepdims=True)
    var = ((v - m) ** 2).mean(axis=-1, keepdims=True)
    return (v - m) * lax.rsqrt(var + eps)


def _a1_body(x_ref, xn_ref, stats_ref):
    i = pl.program_id(0)
    xn = _rln(x_ref[...])
    xn_ref[...] = xn

    @pl.when(i == 0)
    def _():
        stats_ref[...] = jnp.zeros_like(stats_ref)

    s = xn.sum(axis=0)
    ss = (xn * xn).sum(axis=0)
    blk = jnp.concatenate([s[None, :], ss[None, :]], axis=0)
    stats_ref[0:2, :] += blk


def _a1(x_pad):
    return pl.pallas_call(
        _a1_body,
        grid=(GN,),
        in_specs=[pl.BlockSpec((BN, D), lambda i: (i, 0))],
        out_specs=[
            pl.BlockSpec((BN, D), lambda i: (i, 0)),
            pl.BlockSpec((8, D), lambda i: (0, 0)),
        ],
        out_shape=[
            jax.ShapeDtypeStruct((NP, D), jnp.float32),
            jax.ShapeDtypeStruct((8, D), jnp.float32),
        ],
    )(x_pad)


def _a2_body(xn_ref, stats_ref, nz_ref, t_ref, te_ref, w_ref, b_ref, h0_ref):
    xn = xn_ref[...]
    s = stats_ref[0, :]
    ss = stats_ref[1, :]
    miu = s / N
    var = (ss - N * miu * miu) / (N - 1)
    std = jnp.sqrt(var)

    nz = _rln(nz_ref[...])
    nz = nz * std[None, :] + miu[None, :]
    nz = jnp.sign(xn) * jnp.abs(nz)

    kk = lax.broadcasted_iota(jnp.int32, (TP, 1), 0)
    r = kk.astype(jnp.float32)
    betas = 1e-4 + r * ((0.02 - 1e-4) / (T - 1))
    alphas = jnp.where(kk < T, 1.0 - betas, 1.0)
    la = jnp.log(alphas)
    ir = lax.broadcasted_iota(jnp.int32, (TP, TP), 0)
    ic = lax.broadcasted_iota(jnp.int32, (TP, TP), 1)
    tri = (ir >= ic).astype(jnp.float32)
    cum = jnp.dot(tri, la, preferred_element_type=jnp.float32)
    ab = jnp.exp(cum)
    sa = jnp.sqrt(ab)
    sb = jnp.sqrt(jnp.maximum(1.0 - ab, 0.0))
    sasb = jnp.concatenate([sa, sb], axis=1)

    t3 = t_ref[...]
    oh = (lax.broadcasted_iota(jnp.int32, (8, 128, TP), 2)
          == t3[:, :, None]).astype(jnp.float32).reshape(BN, TP)
    g = jnp.dot(oh, sasb, preferred_element_type=jnp.float32)
    te = jnp.dot(oh, te_ref[...], preferred_element_type=jnp.float32)

    x_t = g[:, 0:1] * xn + g[:, 1:2] * nz
    h0 = jax.nn.relu(jnp.dot(x_t, w_ref[...], preferred_element_type=jnp.float32)
                     + b_ref[0, :][None, :]) + te
    h0_ref[...] = h0


def _a2(xn, stats, nz_pad, t_pad, te_pad, W_in, b_in8):
    return pl.pallas_call(
        _a2_body,
        grid=(GN,),
        in_specs=[
            pl.BlockSpec((BN, D), lambda i: (i, 0)),
            pl.BlockSpec((8, D), lambda i: (0, 0)),
            pl.BlockSpec((BN, D), lambda i: (i, 0)),
            pl.BlockSpec((8, 128), lambda i: (i, 0)),
            pl.BlockSpec((TP, H), lambda i: (0, 0)),
            pl.BlockSpec((D, H), lambda i: (0, 0)),
            pl.BlockSpec((8, H), lambda i: (0, 0)),
        ],
        out_specs=pl.BlockSpec((BN, H), lambda i: (i, 0)),
        out_shape=jax.ShapeDtypeStruct((NP, H), jnp.float32),
    )(xn, stats, nz_pad, t_pad, te_pad, W_in, b_in8)


def _b_body(h_ref, w_ref, alm_ref, arm_ref, zlo_ref, zhi_ref, elr_ref, mx_ref):
    i = pl.program_id(0)
    h = h_ref[...]
    z = jnp.dot(h, w_ref[...], preferred_element_type=jnp.float32)
    zlo_ref[...] = z[:, 0:128]
    zhi_ref[...] = z[:, 128:256]
    el = jnp.dot(z, alm_ref[...], preferred_element_type=jnp.float32)
    er = jnp.dot(z, arm_ref[...], preferred_element_type=jnp.float32)
    el_t = el.T
    er_t = er.T
    elr_ref[...] = jnp.concatenate([el_t[0:4, :], er_t[0:4, :]], axis=0)

    @pl.when(i == 0)
    def _():
        mx_ref[...] = jnp.full_like(mx_ref, -1e30)

    blkmax = jnp.max(el_t[0:8, :], axis=1, keepdims=True)
    mx_ref[...] = jnp.maximum(mx_ref[...], blkmax + jnp.zeros((8, 128), jnp.float32))


def _b(h, W, al_mat, ar_mat):
    return pl.pallas_call(
        _b_body,
        grid=(GN,),
        in_specs=[
            pl.BlockSpec((BN, H), lambda i: (i, 0)),
            pl.BlockSpec((H, H), lambda i: (0, 0)),
            pl.BlockSpec((H, 128), lambda i: (0, 0)),
            pl.BlockSpec((H, 128), lambda i: (0, 0)),
        ],
        out_specs=[
            pl.BlockSpec((BN, 128), lambda i: (i, 0)),
            pl.BlockSpec((BN, 128), lambda i: (i, 0)),
            pl.BlockSpec((8, BN), lambda i: (0, i)),
            pl.BlockSpec((8, 128), lambda i: (0, 0)),
        ],
        out_shape=[
            jax.ShapeDtypeStruct((NP, 128), jnp.float32),
            jax.ShapeDtypeStruct((NP, 128), jnp.float32),
            jax.ShapeDtypeStruct((8, NP), jnp.float32),
            jax.ShapeDtypeStruct((8, 128), jnp.float32),
        ],
    )(h, W, al_mat, ar_mat)


def _edge_phase(zlo, zhi, elr, mx, src, dst):
    el = elr[0:4, :].T
    er = elr[4:8, :].T
    maxel = mx[0:4, 0]
    z = jnp.concatenate([zlo, zhi], axis=1)
    e = jax.nn.leaky_relu(el[src] + er[dst], negative_slope=0.2)
    m = jax.nn.leaky_relu(maxel[None, :] + er, negative_slope=0.2)
    ee = jnp.exp(e - m[dst])
    den = jax.ops.segment_sum(ee, dst, num_segments=NP)
    msg = z[src].reshape(E, NHEAD, HD) * ee[:, :, None]
    num = jax.ops.segment_sum(msg, dst, num_segments=NP).reshape(NP, H)
    return num[:, 0:128], num[:, 128:256], den.T


def _c_body(nlo_ref, nhi_ref, den_ref, h_ref, hn_ref):
    num = jnp.concatenate([nlo_ref[...], nhi_ref[...]], axis=1)
    den = den_ref[...].T[:, 0:4]
    dd = jnp.broadcast_to(den[:, :, None], (BN, NHEAD, HD)).reshape(BN, H)
    hn_ref[...] = jax.nn.relu(num / (dd + 1e-9) + h_ref[...])


def _c(nlo, nhi, den, h):
    return pl.pallas_call(
        _c_body,
        grid=(GN,),
        in_specs=[
            pl.BlockSpec((BN, 128), lambda i: (i, 0)),
            pl.BlockSpec((BN, 128), lambda i: (i, 0)),
            pl.BlockSpec((8, BN), lambda i: (0, i)),
            pl.BlockSpec((BN, H), lambda i: (i, 0)),
        ],
        out_specs=pl.BlockSpec((BN, H), lambda i: (i, 0)),
        out_shape=jax.ShapeDtypeStruct((NP, H), jnp.float32),
    )(nlo, nhi, den, h)


def _d_body(h_ref, w_ref, b_ref, xn_ref, loss_ref):
    i = pl.program_id(0)
    h = h_ref[...]
    out = jnp.dot(h, w_ref[...], preferred_element_type=jnp.float32) + b_ref[0, :][None, :]
    xn = xn_ref[...]
    no = jnp.sqrt((out * out).sum(axis=1, keepdims=True))
    nx = jnp.sqrt((xn * xn).sum(axis=1, keepdims=True))
    c = ((out / (no + 1e-12)) * (xn / (nx + 1e-12))).sum(axis=1)
    rid = i * BN + lax.broadcasted_iota(jnp.int32, (BN,), 0)
    part = jnp.where(rid < N, (1.0 - c) ** 2, 0.0).sum()

    @pl.when(i == 0)
    def _():
        loss_ref[0, 0] = 0.0

    loss_ref[0, 0] += part


def _d(h, W_out, b_out8, xn):
    return pl.pallas_call(
        _d_body,
        grid=(GN,),
        in_specs=[
            pl.BlockSpec((BN, H), lambda i: (i, 0)),
            pl.BlockSpec((H, D), lambda i: (0, 0)),
            pl.BlockSpec((8, D), lambda i: (0, 0)),
            pl.BlockSpec((BN, D), lambda i: (i, 0)),
        ],
        out_specs=pl.BlockSpec(memory_space=pltpu.SMEM),
        out_shape=jax.ShapeDtypeStruct((1, 1), jnp.float32),
    )(h, W_out, b_out8, xn)


def _head_mat(a):
    col = jnp.arange(128)[None, :]
    row_head = (jnp.arange(H) // HD)[:, None]
    return jnp.where(col == row_head, a.reshape(H)[:, None], 0.0).astype(jnp.float32)


def kernel(x, edge_index, t, noise, W_in, b_in, gat_W, gat_al, gat_ar, W_out, b_out, time_emb):
    src = edge_index[0]
    dst = edge_index[1]
    pad = NP - N
    x_pad = jnp.pad(x, ((0, pad), (0, 0)))
    nz_pad = jnp.pad(noise, ((0, pad), (0, 0)))
    t_pad = jnp.pad(t, (0, pad)).reshape(80, 128)
    te_pad = jnp.pad(time_emb, ((0, TP - T), (0, 0)))
    b_in8 = jnp.broadcast_to(b_in[None, :], (8, H))
    b_out8 = jnp.broadcast_to(b_out[None, :], (8, D))

    xn, stats = _a1(x_pad)
    h = _a2(xn, stats, nz_pad, t_pad, te_pad, W_in, b_in8)

    for l in range(L):
        zlo, zhi, elr, mx = _b(h, gat_W[l], _head_mat(gat_al[l]), _head_mat(gat_ar[l]))
        nlo, nhi, den = _edge_phase(zlo, zhi, elr, mx, src, dst)
        h = _c(nlo, nhi, den, h)

    loss = _d(h, W_out, b_out8, xn)
    return loss[0, 0] / N


# speedup vs baseline: 40.9404x; 39.4057x over previous
import functools
import jax
import jax.numpy as jnp
from jax import lax
from jax.experimental import pallas as pl
from jax.experimental.pallas import tpu as pltpu
from jax.experimental.pallas import tpu_sc as plsc

N = 10000
D = 128
E = 320000
H = 256
L = 2
NHEAD = 4
HD = 64
T = 1000

NP = 10240
TP = 1024
BN = 1024
GN = NP // BN


def _rln(v, eps=1e-5):
    m = v.mean(axis=-1, keepdims=True)
    var = ((v - m) ** 2).mean(axis=-1, keepdims=True)
    return (v - m) * lax.rsqrt(var + eps)


def _a1_body(x_ref, xn_ref, stats_ref):
    i = pl.program_id(0)
    xn = _rln(x_ref[...])
    xn_ref[...] = xn

    @pl.when(i == 0)
    def _():
        stats_ref[...] = jnp.zeros_like(stats_ref)

    s = xn.sum(axis=0)
    ss = (xn * xn).sum(axis=0)
    blk = jnp.concatenate([s[None, :], ss[None, :]], axis=0)
    stats_ref[0:2, :] += blk


def _a1(x_pad):
    return pl.pallas_call(
        _a1_body,
        grid=(GN,),
        in_specs=[pl.BlockSpec((BN, D), lambda i: (i, 0))],
        out_specs=[
            pl.BlockSpec((BN, D), lambda i: (i, 0)),
            pl.BlockSpec((8, D), lambda i: (0, 0)),
        ],
        out_shape=[
            jax.ShapeDtypeStruct((NP, D), jnp.float32),
            jax.ShapeDtypeStruct((8, D), jnp.float32),
        ],
    )(x_pad)


def _a2_body(xn_ref, stats_ref, nz_ref, t_ref, te_ref, w_ref, b_ref, h0_ref):
    xn = xn_ref[...]
    s = stats_ref[0, :]
    ss = stats_ref[1, :]
    miu = s / N
    var = (ss - N * miu * miu) / (N - 1)
    std = jnp.sqrt(var)

    nz = _rln(nz_ref[...])
    nz = nz * std[None, :] + miu[None, :]
    nz = jnp.sign(xn) * jnp.abs(nz)

    kk = lax.broadcasted_iota(jnp.int32, (TP, 1), 0)
    r = kk.astype(jnp.float32)
    betas = 1e-4 + r * ((0.02 - 1e-4) / (T - 1))
    alphas = jnp.where(kk < T, 1.0 - betas, 1.0)
    la = jnp.log(alphas)
    ir = lax.broadcasted_iota(jnp.int32, (TP, TP), 0)
    ic = lax.broadcasted_iota(jnp.int32, (TP, TP), 1)
    tri = (ir >= ic).astype(jnp.float32)
    cum = jnp.dot(tri, la, preferred_element_type=jnp.float32)
    ab = jnp.exp(cum)
    sa = jnp.sqrt(ab)
    sb = jnp.sqrt(jnp.maximum(1.0 - ab, 0.0))
    sasb = jnp.concatenate([sa, sb], axis=1)

    t3 = t_ref[...]
    oh = (lax.broadcasted_iota(jnp.int32, (8, 128, TP), 2)
          == t3[:, :, None]).astype(jnp.float32).reshape(BN, TP)
    g = jnp.dot(oh, sasb, preferred_element_type=jnp.float32)
    te = jnp.dot(oh, te_ref[...], preferred_element_type=jnp.float32)

    x_t = g[:, 0:1] * xn + g[:, 1:2] * nz
    h0 = jax.nn.relu(jnp.dot(x_t, w_ref[...], preferred_element_type=jnp.float32)
                     + b_ref[0, :][None, :]) + te
    h0_ref[...] = h0


def _a2(xn, stats, nz_pad, t_pad, te_pad, W_in, b_in8):
    return pl.pallas_call(
        _a2_body,
        grid=(GN,),
        in_specs=[
            pl.BlockSpec((BN, D), lambda i: (i, 0)),
            pl.BlockSpec((8, D), lambda i: (0, 0)),
            pl.BlockSpec((BN, D), lambda i: (i, 0)),
            pl.BlockSpec((8, 128), lambda i: (i, 0)),
            pl.BlockSpec((TP, H), lambda i: (0, 0)),
            pl.BlockSpec((D, H), lambda i: (0, 0)),
            pl.BlockSpec((8, H), lambda i: (0, 0)),
        ],
        out_specs=pl.BlockSpec((BN, H), lambda i: (i, 0)),
        out_shape=jax.ShapeDtypeStruct((NP, H), jnp.float32),
    )(xn, stats, nz_pad, t_pad, te_pad, W_in, b_in8)


def _b_body(h_ref, w_ref, alm_ref, arm_ref, zp_ref, elr_ref, mx_ref):
    i = pl.program_id(0)
    h = h_ref[...]
    z = jnp.dot(h, w_ref[...], preferred_element_type=jnp.float32)
    zp_ref[0, :, :] = z[:, 0:128]
    zp_ref[1, :, :] = z[:, 128:256]
    el = jnp.dot(z, alm_ref[...], preferred_element_type=jnp.float32)
    er = jnp.dot(z, arm_ref[...], preferred_element_type=jnp.float32)
    el_t = el.T
    er_t = er.T
    elr_ref[...] = jnp.concatenate([el_t[0:4, :], er_t[0:4, :]], axis=0)

    @pl.when(i == 0)
    def _():
        mx_ref[...] = jnp.full_like(mx_ref, -1e30)

    blkmax = jnp.max(el_t[0:8, :], axis=1, keepdims=True)
    mx_ref[...] = jnp.maximum(mx_ref[...], blkmax + jnp.zeros((8, 128), jnp.float32))


def _b(h, W, al_mat, ar_mat):
    return pl.pallas_call(
        _b_body,
        grid=(GN,),
        in_specs=[
            pl.BlockSpec((BN, H), lambda i: (i, 0)),
            pl.BlockSpec((H, H), lambda i: (0, 0)),
            pl.BlockSpec((H, 128), lambda i: (0, 0)),
            pl.BlockSpec((H, 128), lambda i: (0, 0)),
        ],
        out_specs=[
            pl.BlockSpec((2, BN, 128), lambda i: (0, i, 0)),
            pl.BlockSpec((8, BN), lambda i: (0, i)),
            pl.BlockSpec((8, 128), lambda i: (0, 0)),
        ],
        out_shape=[
            jax.ShapeDtypeStruct((2, NP, 128), jnp.float32),
            jax.ShapeDtypeStruct((8, NP), jnp.float32),
            jax.ShapeDtypeStruct((8, 128), jnp.float32),
        ],
    )(h, W, al_mat, ar_mat)


EC = E // 16
CB = 800
PB = 160
NROW = NP // 16
DR = NP // 128


def _sc_edge_body(zp_hbm, elr_hbm, mx_hbm, src_hbm, dst_hbm,
                  num_hbm, den_hbm, ee_hbm,
                  el_v, er_v, s1_v, d1_v, eec_v,
                  sp_v, dp_v, ee0c_v, ee1c_v, rows_v, mx_v, num_sp, sem):
    c = lax.axis_index("c")
    s = lax.axis_index("s")
    ebase = s * EC

    def _z1(i, _):
        rows_v[i // 8, pl.ds((i % 8) * 16, 16)] = jnp.zeros((16,), jnp.float32)
        return 0
    lax.fori_loop(0, PB * 8, _z1, 0)
    for j in range(4):
        pltpu.sync_copy(rows_v, num_sp.at[pl.ds(s * NROW + j * PB, PB)])

    for hh in range(2):
        h = 2 * c + hh
        pltpu.sync_copy(elr_hbm.at[pl.ds(h * NP, NP)], el_v)
        pltpu.sync_copy(elr_hbm.at[pl.ds((4 + h) * NP, NP)], er_v)
        pltpu.sync_copy(mx_hbm.at[pl.ds(h * 128, 16)], mx_v)

        def _zd(i, _):
            rows_v[i // 8, pl.ds((i % 8) * 16, 16)] = jnp.zeros((16,), jnp.float32)
            return 0
        lax.fori_loop(0, DR * 8, _zd, 0)

        mxb = mx_v[...]

        def _chunk1(ci, _):
            pltpu.sync_copy(src_hbm.at[pl.ds(ebase + ci * CB, CB)], s1_v)
            pltpu.sync_copy(dst_hbm.at[pl.ds(ebase + ci * CB, CB)], d1_v)

            def _grp(j, _):
                s16 = s1_v[pl.ds(j * 16, 16)]
                d16 = d1_v[pl.ds(j * 16, 16)]
                gel = plsc.load_gather(el_v, [s16])
                ger = plsc.load_gather(er_v, [d16])
                a = gel + ger
                e = jnp.where(a > 0, a, 0.2 * a)
                u = mxb + ger
                u = jnp.where(u > 0, u, 0.2 * u)
                ee = jnp.exp(e - u)
                r16 = lax.shift_right_logical(d16, 7)
                c16 = lax.bitwise_and(d16, 127)
                plsc.addupdate_scatter(rows_v, [r16, c16], ee)
                eec_v[pl.ds(j * 16, 16)] = ee
                return 0
            lax.fori_loop(0, CB // 16, _grp, 0)
            pltpu.sync_copy(eec_v, ee_hbm.at[pl.ds((2 * c + hh) * E + ebase + ci * CB, CB)])
            return 0
        lax.fori_loop(0, EC // CB, _chunk1, 0)
        pltpu.sync_copy(rows_v.at[pl.ds(0, DR)], den_hbm.at[c, s, hh])

    plsc.subcore_barrier()
    zview = zp_hbm.at[c]

    def _chunk2(ci, _):
        pltpu.sync_copy(src_hbm.at[pl.ds(ebase + ci * PB, PB)], sp_v)
        pltpu.sync_copy(dst_hbm.at[pl.ds(ebase + ci * PB, PB)], dp_v)
        pltpu.sync_copy(ee_hbm.at[pl.ds(2 * c * E + ebase + ci * PB, PB)], ee0c_v)
        pltpu.sync_copy(ee_hbm.at[pl.ds((2 * c + 1) * E + ebase + ci * PB, PB)], ee1c_v)
        pltpu.async_copy(zview.at[sp_v], rows_v, sem).wait()

        def _scale(k, _):
            i16 = jnp.broadcast_to(k, (16,))
            b0 = plsc.load_gather(ee0c_v, [i16])
            b1 = plsc.load_gather(ee1c_v, [i16])
            for g in range(4):
                rows_v[k, pl.ds(g * 16, 16)] = rows_v[k, pl.ds(g * 16, 16)] * b0
            for g in range(4, 8):
                rows_v[k, pl.ds(g * 16, 16)] = rows_v[k, pl.ds(g * 16, 16)] * b1
            return 0
        lax.fori_loop(0, PB, _scale, 0)
        pltpu.sync_copy(rows_v, num_sp.at[dp_v], add=True)
        return 0
    lax.fori_loop(0, EC // PB, _chunk2, 0)

    plsc.subcore_barrier()
    for j in range(4):
        pltpu.sync_copy(num_sp.at[pl.ds(s * NROW + j * PB, PB)],
                        num_hbm.at[c].at[pl.ds(s * NROW + j * PB, PB)])


def _edge_phase(zp, elr, mx, src, dst):
    mesh = plsc.VectorSubcoreMesh(core_axis_name="c", subcore_axis_name="s")
    f = pl.kernel(
        _sc_edge_body,
        out_type=[
            jax.ShapeDtypeStruct((2, NP, 128), jnp.float32),
            jax.ShapeDtypeStruct((2, 16, 2, DR, 128), jnp.float32),
            jax.ShapeDtypeStruct((2 * 2 * E,), jnp.float32),
        ],
        mesh=mesh,
        compiler_params=pltpu.CompilerParams(needs_layout_passes=False),
        scratch_types=[
            pltpu.VMEM((NP,), jnp.float32),
            pltpu.VMEM((NP,), jnp.float32),
            pltpu.VMEM((CB,), jnp.int32),
            pltpu.VMEM((CB,), jnp.int32),
            pltpu.VMEM((CB,), jnp.float32),
            pltpu.VMEM((PB,), jnp.int32),
            pltpu.VMEM((PB,), jnp.int32),
            pltpu.VMEM((PB,), jnp.float32),
            pltpu.VMEM((PB,), jnp.float32),
            pltpu.VMEM((PB, 128), jnp.float32),
            pltpu.VMEM((16,), jnp.float32),
            pltpu.VMEM_SHARED((NP, 128), jnp.float32),
            pltpu.SemaphoreType.DMA,
        ],
    )
    num, den, _ee = f(zp, elr.reshape(8 * NP), mx.reshape(1024), src, dst)
    return num, den.reshape(2, 16, 2, NP)


def _c_body(num_ref, den_ref, h_ref, hn_ref):
    num = jnp.concatenate([num_ref[0], num_ref[1]], axis=1)
    dp = den_ref[...].reshape(2, 16, 2, BN).sum(axis=1)
    den = dp.reshape(4, BN).T
    dd = jnp.broadcast_to(den[:, :, None], (BN, NHEAD, HD)).reshape(BN, H)
    hn_ref[...] = jax.nn.relu(num / (dd + 1e-9) + h_ref[...])


def _c(num, den, h):
    return pl.pallas_call(
        _c_body,
        grid=(GN,),
        in_specs=[
            pl.BlockSpec((2, BN, 128), lambda i: (0, i, 0)),
            pl.BlockSpec((64, BN), lambda i: (0, i)),
            pl.BlockSpec((BN, H), lambda i: (i, 0)),
        ],
        out_specs=pl.BlockSpec((BN, H), lambda i: (i, 0)),
        out_shape=jax.ShapeDtypeStruct((NP, H), jnp.float32),
    )(num, den.reshape(64, NP), h)


def _d_body(h_ref, w_ref, b_ref, xn_ref, loss_ref):
    i = pl.program_id(0)
    h = h_ref[...]
    out = jnp.dot(h, w_ref[...], preferred_element_type=jnp.float32) + b_ref[0, :][None, :]
    xn = xn_ref[...]
    no = jnp.sqrt((out * out).sum(axis=1, keepdims=True))
    nx = jnp.sqrt((xn * xn).sum(axis=1, keepdims=True))
    c = ((out / (no + 1e-12)) * (xn / (nx + 1e-12))).sum(axis=1)
    rid = i * BN + lax.broadcasted_iota(jnp.int32, (BN,), 0)
    part = jnp.where(rid < N, (1.0 - c) ** 2, 0.0).sum()

    @pl.when(i == 0)
    def _():
        loss_ref[0, 0] = 0.0

    loss_ref[0, 0] += part


def _d(h, W_out, b_out8, xn):
    return pl.pallas_call(
        _d_body,
        grid=(GN,),
        in_specs=[
            pl.BlockSpec((BN, H), lambda i: (i, 0)),
            pl.BlockSpec((H, D), lambda i: (0, 0)),
            pl.BlockSpec((8, D), lambda i: (0, 0)),
            pl.BlockSpec((BN, D), lambda i: (i, 0)),
        ],
        out_specs=pl.BlockSpec(memory_space=pltpu.SMEM),
        out_shape=jax.ShapeDtypeStruct((1, 1), jnp.float32),
    )(h, W_out, b_out8, xn)


def _head_mat(a):
    col = jnp.arange(128)[None, :]
    row_head = (jnp.arange(H) // HD)[:, None]
    return jnp.where(col == row_head, a.reshape(H)[:, None], 0.0).astype(jnp.float32)


def kernel(x, edge_index, t, noise, W_in, b_in, gat_W, gat_al, gat_ar, W_out, b_out, time_emb):
    src = edge_index[0]
    dst = edge_index[1]
    pad = NP - N
    x_pad = jnp.pad(x, ((0, pad), (0, 0)))
    nz_pad = jnp.pad(noise, ((0, pad), (0, 0)))
    t_pad = jnp.pad(t, (0, pad)).reshape(80, 128)
    te_pad = jnp.pad(time_emb, ((0, TP - T), (0, 0)))
    b_in8 = jnp.broadcast_to(b_in[None, :], (8, H))
    b_out8 = jnp.broadcast_to(b_out[None, :], (8, D))

    xn, stats = _a1(x_pad)
    h = _a2(xn, stats, nz_pad, t_pad, te_pad, W_in, b_in8)

    for l in range(L):
        zp, elr, mx = _b(h, gat_W[l], _head_mat(gat_al[l]), _head_mat(gat_ar[l]))
        num, den = _edge_phase(zp, elr, mx, src, dst)
        h = _c(num, den, h)

    loss = _d(h, W_out, b_out8, xn)
    return loss[0, 0] / N


# speedup vs baseline: 71.0501x; 1.7355x over previous
import functools
import jax
import jax.numpy as jnp
from jax import lax
from jax.experimental import pallas as pl
from jax.experimental.pallas import tpu as pltpu
from jax.experimental.pallas import tpu_sc as plsc

N = 10000
D = 128
E = 320000
H = 256
L = 2
NHEAD = 4
HD = 64
T = 1000

NP = 10240
TP = 1024
BN = 1024
GN = NP // BN


def _rln(v, eps=1e-5):
    m = v.mean(axis=-1, keepdims=True)
    var = ((v - m) ** 2).mean(axis=-1, keepdims=True)
    return (v - m) * lax.rsqrt(var + eps)


def _a1_body(x_ref, xn_ref, stats_ref):
    i = pl.program_id(0)
    xn = _rln(x_ref[...])
    xn_ref[...] = xn

    @pl.when(i == 0)
    def _():
        stats_ref[...] = jnp.zeros_like(stats_ref)

    s = xn.sum(axis=0)
    ss = (xn * xn).sum(axis=0)
    blk = jnp.concatenate([s[None, :], ss[None, :]], axis=0)
    stats_ref[0:2, :] += blk


def _a1(x_pad):
    return pl.pallas_call(
        _a1_body,
        grid=(GN,),
        in_specs=[pl.BlockSpec((BN, D), lambda i: (i, 0))],
        out_specs=[
            pl.BlockSpec((BN, D), lambda i: (i, 0)),
            pl.BlockSpec((8, D), lambda i: (0, 0)),
        ],
        out_shape=[
            jax.ShapeDtypeStruct((NP, D), jnp.float32),
            jax.ShapeDtypeStruct((8, D), jnp.float32),
        ],
    )(x_pad)


def _a2_body(xn_ref, stats_ref, nz_ref, t_ref, te_ref, w_ref, b_ref, h0_ref):
    xn = xn_ref[...]
    s = stats_ref[0, :]
    ss = stats_ref[1, :]
    miu = s / N
    var = (ss - N * miu * miu) / (N - 1)
    std = jnp.sqrt(var)

    nz = _rln(nz_ref[...])
    nz = nz * std[None, :] + miu[None, :]
    nz = jnp.sign(xn) * jnp.abs(nz)

    kk = lax.broadcasted_iota(jnp.int32, (TP, 1), 0)
    r = kk.astype(jnp.float32)
    betas = 1e-4 + r * ((0.02 - 1e-4) / (T - 1))
    alphas = jnp.where(kk < T, 1.0 - betas, 1.0)
    la = jnp.log(alphas)
    ir = lax.broadcasted_iota(jnp.int32, (TP, TP), 0)
    ic = lax.broadcasted_iota(jnp.int32, (TP, TP), 1)
    tri = (ir >= ic).astype(jnp.float32)
    cum = jnp.dot(tri, la, preferred_element_type=jnp.float32)
    ab = jnp.exp(cum)
    sa = jnp.sqrt(ab)
    sb = jnp.sqrt(jnp.maximum(1.0 - ab, 0.0))
    sasb = jnp.concatenate([sa, sb], axis=1)

    t3 = t_ref[...]
    oh = (lax.broadcasted_iota(jnp.int32, (8, 128, TP), 2)
          == t3[:, :, None]).astype(jnp.float32).reshape(BN, TP)
    g = jnp.dot(oh, sasb, preferred_element_type=jnp.float32)
    te = jnp.dot(oh, te_ref[...], preferred_element_type=jnp.float32)

    x_t = g[:, 0:1] * xn + g[:, 1:2] * nz
    h0 = jax.nn.relu(jnp.dot(x_t, w_ref[...], preferred_element_type=jnp.float32)
                     + b_ref[0, :][None, :]) + te
    h0_ref[...] = h0


def _a2(xn, stats, nz_pad, t_pad, te_pad, W_in, b_in8):
    return pl.pallas_call(
        _a2_body,
        grid=(GN,),
        in_specs=[
            pl.BlockSpec((BN, D), lambda i: (i, 0)),
            pl.BlockSpec((8, D), lambda i: (0, 0)),
            pl.BlockSpec((BN, D), lambda i: (i, 0)),
            pl.BlockSpec((8, 128), lambda i: (i, 0)),
            pl.BlockSpec((TP, H), lambda i: (0, 0)),
            pl.BlockSpec((D, H), lambda i: (0, 0)),
            pl.BlockSpec((8, H), lambda i: (0, 0)),
        ],
        out_specs=pl.BlockSpec((BN, H), lambda i: (i, 0)),
        out_shape=jax.ShapeDtypeStruct((NP, H), jnp.float32),
    )(xn, stats, nz_pad, t_pad, te_pad, W_in, b_in8)


def _b_body(h_ref, w_ref, alm_ref, arm_ref, zp_ref, elr_ref, mx_ref):
    i = pl.program_id(0)
    h = h_ref[...]
    z = jnp.dot(h, w_ref[...], preferred_element_type=jnp.float32)
    zp_ref[0, :, :] = z[:, 0:128]
    zp_ref[1, :, :] = z[:, 128:256]
    el = jnp.dot(z, alm_ref[...], preferred_element_type=jnp.float32)
    er = jnp.dot(z, arm_ref[...], preferred_element_type=jnp.float32)
    el_t = el.T
    er_t = er.T
    elr_ref[...] = jnp.concatenate([el_t[0:4, :], er_t[0:4, :]], axis=0)

    @pl.when(i == 0)
    def _():
        mx_ref[...] = jnp.full_like(mx_ref, -1e30)

    blkmax = jnp.max(el_t[0:8, :], axis=1, keepdims=True)
    mx_ref[...] = jnp.maximum(mx_ref[...], blkmax + jnp.zeros((8, 128), jnp.float32))


def _b(h, W, al_mat, ar_mat):
    return pl.pallas_call(
        _b_body,
        grid=(GN,),
        in_specs=[
            pl.BlockSpec((BN, H), lambda i: (i, 0)),
            pl.BlockSpec((H, H), lambda i: (0, 0)),
            pl.BlockSpec((H, 128), lambda i: (0, 0)),
            pl.BlockSpec((H, 128), lambda i: (0, 0)),
        ],
        out_specs=[
            pl.BlockSpec((2, BN, 128), lambda i: (0, i, 0)),
            pl.BlockSpec((8, BN), lambda i: (0, i)),
            pl.BlockSpec((8, 128), lambda i: (0, 0)),
        ],
        out_shape=[
            jax.ShapeDtypeStruct((2, NP, 128), jnp.float32),
            jax.ShapeDtypeStruct((8, NP), jnp.float32),
            jax.ShapeDtypeStruct((8, 128), jnp.float32),
        ],
    )(h, W, al_mat, ar_mat)


EC = E // 16
CB = 800
PB = 80
NROW = NP // 16
DR = NP // 128


NCH = EC // PB


def _sc_edge_body(zp_hbm, elr_hbm, mx_hbm, src_hbm, dst_hbm,
                  num_hbm, den_hbm, ee_hbm,
                  el_v, er_v, s1_v, d1_v, eec_v,
                  sp_v, dp_v, e0_v, e1_v, rows_v, mx_v, num_sp,
                  st0, st1, st2, st3, g0, g1, ss0, ss1):
    c = lax.axis_index("c")
    s = lax.axis_index("s")
    ebase = s * EC
    stsem = [st0, st1, st2, st3]
    gsem = [g0, g1]
    ssem = [ss0, ss1]

    def _z1(i, _):
        rows_v[0, i // 8, pl.ds((i % 8) * 16, 16)] = jnp.zeros((16,), jnp.float32)
        return 0
    lax.fori_loop(0, PB * 8, _z1, 0)
    for j in range(8):
        pltpu.sync_copy(rows_v.at[0], num_sp.at[pl.ds(s * NROW + j * PB, PB)])

    zero16 = jnp.zeros((16,), jnp.int32)
    for hh in range(2):
        h = 2 * c + hh
        pltpu.sync_copy(elr_hbm.at[pl.ds(h * NP, NP)], el_v)
        pltpu.sync_copy(elr_hbm.at[pl.ds((4 + h) * NP, NP)], er_v)
        pltpu.sync_copy(mx_hbm.at[pl.ds(h * 128, 16)], mx_v)

        def _zd(i, _):
            rows_v[0, i // 8, pl.ds((i % 8) * 16, 16)] = jnp.zeros((16,), jnp.float32)
            return 0
        lax.fori_loop(0, DR * 8, _zd, 0)

        mxb = mx_v[...]

        def _chunk1(ci, _):
            pltpu.sync_copy(src_hbm.at[pl.ds(ebase + ci * CB, CB)], s1_v)
            pltpu.sync_copy(dst_hbm.at[pl.ds(ebase + ci * CB, CB)], d1_v)

            def _grp(j, _):
                s16 = s1_v[pl.ds(j * 16, 16)]
                d16 = d1_v[pl.ds(j * 16, 16)]
                gel = plsc.load_gather(el_v, [s16])
                ger = plsc.load_gather(er_v, [d16])
                a = gel + ger
                e = jnp.where(a > 0, a, 0.2 * a)
                u = mxb + ger
                u = jnp.where(u > 0, u, 0.2 * u)
                ee = jnp.exp(e - u)
                r16 = lax.shift_right_logical(d16, 7)
                c16 = lax.bitwise_and(d16, 127)
                plsc.addupdate_scatter(rows_v, [zero16, r16, c16], ee)
                eec_v[pl.ds(j * 16, 16)] = ee
                return 0
            lax.fori_loop(0, CB // 16, _grp, 0)
            pltpu.sync_copy(eec_v, ee_hbm.at[pl.ds((2 * c + hh) * E + ebase + ci * CB, CB)])
            return 0
        lax.fori_loop(0, EC // CB, _chunk1, 0)
        pltpu.sync_copy(rows_v.at[0], den_hbm.at[c, s, hh])

    plsc.subcore_barrier()
    zview = zp_hbm.at[c]

    def _stage(i, r):
        off = ebase + i * PB
        pltpu.async_copy(src_hbm.at[pl.ds(off, PB)], sp_v.at[r], stsem[r])
        pltpu.async_copy(dst_hbm.at[pl.ds(off, PB)], dp_v.at[r], stsem[r])
        pltpu.async_copy(ee_hbm.at[pl.ds(2 * c * E + off, PB)], e0_v.at[r], stsem[r])
        pltpu.async_copy(ee_hbm.at[pl.ds((2 * c + 1) * E + off, PB)], e1_v.at[r], stsem[r])

    def _wait_stage(i, r):
        off = ebase + i * PB
        pltpu.make_async_copy(src_hbm.at[pl.ds(off, PB)], sp_v.at[r], stsem[r]).wait()
        pltpu.make_async_copy(dst_hbm.at[pl.ds(off, PB)], dp_v.at[r], stsem[r]).wait()
        pltpu.make_async_copy(ee_hbm.at[pl.ds(2 * c * E + off, PB)], e0_v.at[r], stsem[r]).wait()
        pltpu.make_async_copy(ee_hbm.at[pl.ds((2 * c + 1) * E + off, PB)], e1_v.at[r], stsem[r]).wait()

    def _wait_scatter(p, r):
        pltpu.make_async_copy(rows_v.at[p], num_sp.at[dp_v.at[r]], ssem[p]).wait()

    def _scale(p, r):
        r16 = jnp.full((16,), r, jnp.int32)

        def body(k, _):
            i16 = jnp.broadcast_to(k, (16,))
            b0 = plsc.load_gather(e0_v, [r16, i16])
            b1 = plsc.load_gather(e1_v, [r16, i16])
            for g in range(4):
                rows_v[p, k, pl.ds(g * 16, 16)] = rows_v[p, k, pl.ds(g * 16, 16)] * b0
            for g in range(4, 8):
                rows_v[p, k, pl.ds(g * 16, 16)] = rows_v[p, k, pl.ds(g * 16, 16)] * b1
            return 0
        lax.fori_loop(0, PB, body, 0)

    def _sub(i, r, p, skip_wait_scatter, skip_stage2):
        q = 1 - p
        _wait_stage(i + 1, (r + 1) % 4)
        if skip_wait_scatter is None:
            _wait_scatter(q, (r + 3) % 4)
        elif skip_wait_scatter is not True:
            @pl.when(skip_wait_scatter)
            def _():
                _wait_scatter(q, (r + 3) % 4)
        pltpu.async_copy(zview.at[sp_v.at[(r + 1) % 4]], rows_v.at[q], gsem[q])
        pltpu.make_async_copy(zview.at[sp_v.at[r]], rows_v.at[p], gsem[p]).wait()
        if not skip_stage2:
            _stage(i + 2, (r + 2) % 4)
        _scale(p, r)
        pltpu.async_copy(rows_v.at[p], num_sp.at[dp_v.at[r]], ssem[p], add=True)

    _stage(0, 0)
    _stage(1, 1)
    _wait_stage(0, 0)
    pltpu.async_copy(zview.at[sp_v.at[0]], rows_v.at[0], gsem[0])

    def _quad(k, _):
        i0 = k * 4
        _sub(i0, 0, 0, skip_wait_scatter=(k > 0), skip_stage2=False)
        _sub(i0 + 1, 1, 1, skip_wait_scatter=None, skip_stage2=False)
        _sub(i0 + 2, 2, 0, skip_wait_scatter=None, skip_stage2=False)
        _sub(i0 + 3, 3, 1, skip_wait_scatter=None, skip_stage2=False)
        return 0
    lax.fori_loop(0, NCH // 4, _quad, 0)

    _wait_stage(NCH - 1, 1)
    _wait_scatter(1, 3)
    pltpu.async_copy(zview.at[sp_v.at[1]], rows_v.at[1], gsem[1])
    pltpu.make_async_copy(zview.at[sp_v.at[0]], rows_v.at[0], gsem[0]).wait()
    _scale(0, 0)
    pltpu.async_copy(rows_v.at[0], num_sp.at[dp_v.at[0]], ssem[0], add=True)
    _wait_scatter(0, 0)
    pltpu.make_async_copy(zview.at[sp_v.at[1]], rows_v.at[1], gsem[1]).wait()
    _scale(1, 1)
    pltpu.async_copy(rows_v.at[1], num_sp.at[dp_v.at[1]], ssem[1], add=True)
    _wait_scatter(1, 1)

    plsc.subcore_barrier()
    for j in range(8):
        pltpu.sync_copy(num_sp.at[pl.ds(s * NROW + j * PB, PB)],
                        num_hbm.at[c].at[pl.ds(s * NROW + j * PB, PB)])


def _edge_phase(zp, elr, mx, src, dst):
    mesh = plsc.VectorSubcoreMesh(core_axis_name="c", subcore_axis_name="s")
    f = pl.kernel(
        _sc_edge_body,
        out_type=[
            jax.ShapeDtypeStruct((2, NP, 128), jnp.float32),
            jax.ShapeDtypeStruct((2, 16, 2, DR, 128), jnp.float32),
            jax.ShapeDtypeStruct((2 * 2 * E,), jnp.float32),
        ],
        mesh=mesh,
        compiler_params=pltpu.CompilerParams(needs_layout_passes=False),
        scratch_types=[
            pltpu.VMEM((NP,), jnp.float32),
            pltpu.VMEM((NP,), jnp.float32),
            pltpu.VMEM((CB,), jnp.int32),
            pltpu.VMEM((CB,), jnp.int32),
            pltpu.VMEM((CB,), jnp.float32),
            pltpu.VMEM((4, PB), jnp.int32),
            pltpu.VMEM((4, PB), jnp.int32),
            pltpu.VMEM((4, PB), jnp.float32),
            pltpu.VMEM((4, PB), jnp.float32),
            pltpu.VMEM((2, PB, 128), jnp.float32),
            pltpu.VMEM((16,), jnp.float32),
            pltpu.VMEM_SHARED((NP, 128), jnp.float32),
            pltpu.SemaphoreType.DMA,
            pltpu.SemaphoreType.DMA,
            pltpu.SemaphoreType.DMA,
            pltpu.SemaphoreType.DMA,
            pltpu.SemaphoreType.DMA,
            pltpu.SemaphoreType.DMA,
            pltpu.SemaphoreType.DMA,
            pltpu.SemaphoreType.DMA,
        ],
    )
    num, den, _ee = f(zp, elr.reshape(8 * NP), mx.reshape(1024), src, dst)
    return num, den.reshape(2, 16, 2, NP)


def _c_body(num_ref, den_ref, h_ref, hn_ref):
    num = jnp.concatenate([num_ref[0], num_ref[1]], axis=1)
    dp = den_ref[...].reshape(2, 16, 2, BN).sum(axis=1)
    den = dp.reshape(4, BN).T
    dd = jnp.broadcast_to(den[:, :, None], (BN, NHEAD, HD)).reshape(BN, H)
    hn_ref[...] = jax.nn.relu(num / (dd + 1e-9) + h_ref[...])


def _c(num, den, h):
    return pl.pallas_call(
        _c_body,
        grid=(GN,),
        in_specs=[
            pl.BlockSpec((2, BN, 128), lambda i: (0, i, 0)),
            pl.BlockSpec((64, BN), lambda i: (0, i)),
            pl.BlockSpec((BN, H), lambda i: (i, 0)),
        ],
        out_specs=pl.BlockSpec((BN, H), lambda i: (i, 0)),
        out_shape=jax.ShapeDtypeStruct((NP, H), jnp.float32),
    )(num, den.reshape(64, NP), h)


def _d_body(h_ref, w_ref, b_ref, xn_ref, loss_ref):
    i = pl.program_id(0)
    h = h_ref[...]
    out = jnp.dot(h, w_ref[...], preferred_element_type=jnp.float32) + b_ref[0, :][None, :]
    xn = xn_ref[...]
    no = jnp.sqrt((out * out).sum(axis=1, keepdims=True))
    nx = jnp.sqrt((xn * xn).sum(axis=1, keepdims=True))
    c = ((out / (no + 1e-12)) * (xn / (nx + 1e-12))).sum(axis=1)
    rid = i * BN + lax.broadcasted_iota(jnp.int32, (BN,), 0)
    part = jnp.where(rid < N, (1.0 - c) ** 2, 0.0).sum()

    @pl.when(i == 0)
    def _():
        loss_ref[0, 0] = 0.0

    loss_ref[0, 0] += part


def _d(h, W_out, b_out8, xn):
    return pl.pallas_call(
        _d_body,
        grid=(GN,),
        in_specs=[
            pl.BlockSpec((BN, H), lambda i: (i, 0)),
            pl.BlockSpec((H, D), lambda i: (0, 0)),
            pl.BlockSpec((8, D), lambda i: (0, 0)),
            pl.BlockSpec((BN, D), lambda i: (i, 0)),
        ],
        out_specs=pl.BlockSpec(memory_space=pltpu.SMEM),
        out_shape=jax.ShapeDtypeStruct((1, 1), jnp.float32),
    )(h, W_out, b_out8, xn)


def _head_mat(a):
    col = jnp.arange(128)[None, :]
    row_head = (jnp.arange(H) // HD)[:, None]
    return jnp.where(col == row_head, a.reshape(H)[:, None], 0.0).astype(jnp.float32)


def kernel(x, edge_index, t, noise, W_in, b_in, gat_W, gat_al, gat_ar, W_out, b_out, time_emb):
    src = edge_index[0]
    dst = edge_index[1]
    pad = NP - N
    x_pad = jnp.pad(x, ((0, pad), (0, 0)))
    nz_pad = jnp.pad(noise, ((0, pad), (0, 0)))
    t_pad = jnp.pad(t, (0, pad)).reshape(80, 128)
    te_pad = jnp.pad(time_emb, ((0, TP - T), (0, 0)))
    b_in8 = jnp.broadcast_to(b_in[None, :], (8, H))
    b_out8 = jnp.broadcast_to(b_out[None, :], (8, D))

    xn, stats = _a1(x_pad)
    h = _a2(xn, stats, nz_pad, t_pad, te_pad, W_in, b_in8)

    for l in range(L):
        zp, elr, mx = _b(h, gat_W[l], _head_mat(gat_al[l]), _head_mat(gat_ar[l]))
        num, den = _edge_phase(zp, elr, mx, src, dst)
        h = _c(num, den, h)

    loss = _d(h, W_out, b_out8, xn)
    return loss[0, 0] / N


# speedup vs baseline: 78.7120x; 1.1078x over previous
import functools
import jax
import jax.numpy as jnp
from jax import lax
from jax.experimental import pallas as pl
from jax.experimental.pallas import tpu as pltpu
from jax.experimental.pallas import tpu_sc as plsc

N = 10000
D = 128
E = 320000
H = 256
L = 2
NHEAD = 4
HD = 64
T = 1000

NP = 10240
TP = 1024
BN = 1024
GN = NP // BN


def _rln(v, eps=1e-5):
    m = v.mean(axis=-1, keepdims=True)
    var = ((v - m) ** 2).mean(axis=-1, keepdims=True)
    return (v - m) * lax.rsqrt(var + eps)


def _a1_body(x_ref, xn_ref, stats_ref):
    i = pl.program_id(0)
    xn = _rln(x_ref[...])
    xn_ref[...] = xn

    @pl.when(i == 0)
    def _():
        stats_ref[...] = jnp.zeros_like(stats_ref)

    s = xn.sum(axis=0)
    ss = (xn * xn).sum(axis=0)
    blk = jnp.concatenate([s[None, :], ss[None, :]], axis=0)
    stats_ref[0:2, :] += blk


def _a1(x_pad):
    return pl.pallas_call(
        _a1_body,
        grid=(GN,),
        in_specs=[pl.BlockSpec((BN, D), lambda i: (i, 0))],
        out_specs=[
            pl.BlockSpec((BN, D), lambda i: (i, 0)),
            pl.BlockSpec((8, D), lambda i: (0, 0)),
        ],
        out_shape=[
            jax.ShapeDtypeStruct((NP, D), jnp.float32),
            jax.ShapeDtypeStruct((8, D), jnp.float32),
        ],
    )(x_pad)


def _a2_body(xn_ref, stats_ref, nz_ref, t_ref, te_ref, w_ref, b_ref, h0_ref):
    xn = xn_ref[...]
    s = stats_ref[0, :]
    ss = stats_ref[1, :]
    miu = s / N
    var = (ss - N * miu * miu) / (N - 1)
    std = jnp.sqrt(var)

    nz = _rln(nz_ref[...])
    nz = nz * std[None, :] + miu[None, :]
    nz = jnp.sign(xn) * jnp.abs(nz)

    kk = lax.broadcasted_iota(jnp.int32, (TP, 1), 0)
    r = kk.astype(jnp.float32)
    betas = 1e-4 + r * ((0.02 - 1e-4) / (T - 1))
    alphas = jnp.where(kk < T, 1.0 - betas, 1.0)
    la = jnp.log(alphas)
    ir = lax.broadcasted_iota(jnp.int32, (TP, TP), 0)
    ic = lax.broadcasted_iota(jnp.int32, (TP, TP), 1)
    tri = (ir >= ic).astype(jnp.float32)
    cum = jnp.dot(tri, la, preferred_element_type=jnp.float32)
    ab = jnp.exp(cum)
    sa = jnp.sqrt(ab)
    sb = jnp.sqrt(jnp.maximum(1.0 - ab, 0.0))
    sasb = jnp.concatenate([sa, sb], axis=1)

    t3 = t_ref[...]
    oh = (lax.broadcasted_iota(jnp.int32, (8, 128, TP), 2)
          == t3[:, :, None]).astype(jnp.float32).reshape(BN, TP)
    g = jnp.dot(oh, sasb, preferred_element_type=jnp.float32)
    te = jnp.dot(oh, te_ref[...], preferred_element_type=jnp.float32)

    x_t = g[:, 0:1] * xn + g[:, 1:2] * nz
    h0 = jax.nn.relu(jnp.dot(x_t, w_ref[...], preferred_element_type=jnp.float32)
                     + b_ref[0, :][None, :]) + te
    h0_ref[...] = h0


def _a2(xn, stats, nz_pad, t_pad, te_pad, W_in, b_in8):
    return pl.pallas_call(
        _a2_body,
        grid=(GN,),
        in_specs=[
            pl.BlockSpec((BN, D), lambda i: (i, 0)),
            pl.BlockSpec((8, D), lambda i: (0, 0)),
            pl.BlockSpec((BN, D), lambda i: (i, 0)),
            pl.BlockSpec((8, 128), lambda i: (i, 0)),
            pl.BlockSpec((TP, H), lambda i: (0, 0)),
            pl.BlockSpec((D, H), lambda i: (0, 0)),
            pl.BlockSpec((8, H), lambda i: (0, 0)),
        ],
        out_specs=pl.BlockSpec((BN, H), lambda i: (i, 0)),
        out_shape=jax.ShapeDtypeStruct((NP, H), jnp.float32),
    )(xn, stats, nz_pad, t_pad, te_pad, W_in, b_in8)


def _b_body(h_ref, w_ref, alm_ref, arm_ref, zp_ref, elr_ref, mx_ref):
    i = pl.program_id(0)
    h = h_ref[...]
    z = jnp.dot(h, w_ref[...], preferred_element_type=jnp.float32)
    zp_ref[0, :, :] = z[:, 0:128]
    zp_ref[1, :, :] = z[:, 128:256]
    el = jnp.dot(z, alm_ref[...], preferred_element_type=jnp.float32)
    er = jnp.dot(z, arm_ref[...], preferred_element_type=jnp.float32)
    el_t = el.T
    er_t = er.T
    elr_ref[...] = jnp.concatenate([el_t[0:4, :], er_t[0:4, :]], axis=0)

    @pl.when(i == 0)
    def _():
        mx_ref[...] = jnp.full_like(mx_ref, -1e30)

    blkmax = jnp.max(el_t[0:8, :], axis=1, keepdims=True)
    mx_ref[...] = jnp.maximum(mx_ref[...], blkmax + jnp.zeros((8, 128), jnp.float32))


def _b(h, W, al_mat, ar_mat):
    return pl.pallas_call(
        _b_body,
        grid=(GN,),
        in_specs=[
            pl.BlockSpec((BN, H), lambda i: (i, 0)),
            pl.BlockSpec((H, H), lambda i: (0, 0)),
            pl.BlockSpec((H, 128), lambda i: (0, 0)),
            pl.BlockSpec((H, 128), lambda i: (0, 0)),
        ],
        out_specs=[
            pl.BlockSpec((2, BN, 128), lambda i: (0, i, 0)),
            pl.BlockSpec((8, BN), lambda i: (0, i)),
            pl.BlockSpec((8, 128), lambda i: (0, 0)),
        ],
        out_shape=[
            jax.ShapeDtypeStruct((2, NP, 128), jnp.float32),
            jax.ShapeDtypeStruct((8, NP), jnp.float32),
            jax.ShapeDtypeStruct((8, 128), jnp.float32),
        ],
    )(h, W, al_mat, ar_mat)


EC = E // 16
CB = 400
PB = 80
NROW = NP // 16
DR = NP // 128


NCH = EC // PB


def _sc_edge_body(zp_hbm, elr_hbm, mx_hbm, src_hbm, dst_hbm,
                  num_hbm, den_hbm, ee_hbm,
                  el_v, er_v, s1a_v, s1b_v, d1a_v, d1b_v, eea_v, eeb_v,
                  sp_v, dp_v, e0_v, e1_v, rows_v, mx_v, num_sp,
                  st0, st1, st2, st3, g0, g1, ss0, ss1):
    c = lax.axis_index("c")
    s = lax.axis_index("s")
    ebase = s * EC
    stsem = [st0, st1, st2, st3]
    gsem = [g0, g1]
    ssem = [ss0, ss1]
    s1l = [s1a_v, s1b_v]
    d1l = [d1a_v, d1b_v]
    eel = [eea_v, eeb_v]

    def _z1(i, _):
        rows_v[0, i // 8, pl.ds((i % 8) * 16, 16)] = jnp.zeros((16,), jnp.float32)
        return 0
    lax.fori_loop(0, PB * 8, _z1, 0)
    for j in range(8):
        pltpu.sync_copy(rows_v.at[0], num_sp.at[pl.ds(s * NROW + j * PB, PB)])

    zero16 = jnp.zeros((16,), jnp.int32)
    NC1 = EC // CB

    for hh in range(2):
        h = 2 * c + hh
        pltpu.sync_copy(elr_hbm.at[pl.ds(h * NP, NP)], el_v)
        pltpu.sync_copy(elr_hbm.at[pl.ds((4 + h) * NP, NP)], er_v)
        pltpu.sync_copy(mx_hbm.at[pl.ds(h * 128, 16)], mx_v)

        def _zd(i, _):
            rows_v[0, i // 8, pl.ds((i % 8) * 16, 16)] = jnp.zeros((16,), jnp.float32)
            return 0
        lax.fori_loop(0, DR * 8, _zd, 0)

        mxb = mx_v[...]

        def _stage1(i, u):
            pltpu.async_copy(src_hbm.at[pl.ds(ebase + i * CB, CB)], s1l[u], stsem[u])
            pltpu.async_copy(dst_hbm.at[pl.ds(ebase + i * CB, CB)], d1l[u], stsem[u])

        def _wait_stage1(i, u):
            pltpu.make_async_copy(src_hbm.at[pl.ds(ebase + i * CB, CB)], s1l[u], stsem[u]).wait()
            pltpu.make_async_copy(dst_hbm.at[pl.ds(ebase + i * CB, CB)], d1l[u], stsem[u]).wait()

        def _ee_off(i):
            return (2 * c + hh) * E + ebase + i * CB

        def _sub1(i, u, wait_wb, last):
            _wait_stage1(i, u)
            if not last:
                @pl.when(i + 1 < NC1)
                def _():
                    _stage1(i + 1, 1 - u)
            if wait_wb is not None:
                @pl.when(wait_wb)
                def _():
                    pltpu.make_async_copy(eel[u], ee_hbm.at[pl.ds(_ee_off(i - 2), CB)],
                                          stsem[2 + u]).wait()

            def _grp(j, _):
                s16 = s1l[u][pl.ds(j * 16, 16)]
                d16 = d1l[u][pl.ds(j * 16, 16)]
                gel = plsc.load_gather(el_v, [s16])
                ger = plsc.load_gather(er_v, [d16])
                a = gel + ger
                e = jnp.where(a > 0, a, 0.2 * a)
                u2 = mxb + ger
                u2 = jnp.where(u2 > 0, u2, 0.2 * u2)
                ee = jnp.exp(e - u2)
                r16 = lax.shift_right_logical(d16, 7)
                c16 = lax.bitwise_and(d16, 127)
                plsc.addupdate_scatter(rows_v, [zero16, r16, c16], ee)
                eel[u][pl.ds(j * 16, 16)] = ee
                return 0
            lax.fori_loop(0, CB // 16, _grp, 0)
            pltpu.async_copy(eel[u], ee_hbm.at[pl.ds(_ee_off(i), CB)], stsem[2 + u])

        _stage1(0, 0)

        def _pair1(k, _):
            i0 = 2 * k
            _sub1(i0, 0, (k > 0), False)
            _sub1(i0 + 1, 1, (k > 0), False)
            return 0
        lax.fori_loop(0, NC1 // 2 - 1, _pair1, 0)
        _sub1(NC1 - 2, 0, True, False)
        _sub1(NC1 - 1, 1, True, True)
        for u in range(2):
            pltpu.make_async_copy(eel[u], ee_hbm.at[pl.ds(_ee_off(NC1 - 2 + u), CB)],
                                  stsem[2 + u]).wait()
        pltpu.sync_copy(rows_v.at[0], den_hbm.at[c, s, hh])

    plsc.subcore_barrier()
    zview = zp_hbm.at[c]

    def _stage(i, r):
        off = ebase + i * PB
        pltpu.async_copy(src_hbm.at[pl.ds(off, PB)], sp_v.at[r], stsem[r])
        pltpu.async_copy(dst_hbm.at[pl.ds(off, PB)], dp_v.at[r], stsem[r])
        pltpu.async_copy(ee_hbm.at[pl.ds(2 * c * E + off, PB)], e0_v.at[r], stsem[r])
        pltpu.async_copy(ee_hbm.at[pl.ds((2 * c + 1) * E + off, PB)], e1_v.at[r], stsem[r])

    def _wait_stage(i, r):
        off = ebase + i * PB
        pltpu.make_async_copy(src_hbm.at[pl.ds(off, PB)], sp_v.at[r], stsem[r]).wait()
        pltpu.make_async_copy(dst_hbm.at[pl.ds(off, PB)], dp_v.at[r], stsem[r]).wait()
        pltpu.make_async_copy(ee_hbm.at[pl.ds(2 * c * E + off, PB)], e0_v.at[r], stsem[r]).wait()
        pltpu.make_async_copy(ee_hbm.at[pl.ds((2 * c + 1) * E + off, PB)], e1_v.at[r], stsem[r]).wait()

    def _wait_scatter(p, r):
        pltpu.make_async_copy(rows_v.at[p], num_sp.at[dp_v.at[r]], ssem[p]).wait()

    def _scale(p, r):
        r16 = jnp.full((16,), r, jnp.int32)

        def body(k, _):
            i16 = jnp.broadcast_to(k, (16,))
            b0 = plsc.load_gather(e0_v, [r16, i16])
            b1 = plsc.load_gather(e1_v, [r16, i16])
            for g in range(4):
                rows_v[p, k, pl.ds(g * 16, 16)] = rows_v[p, k, pl.ds(g * 16, 16)] * b0
            for g in range(4, 8):
                rows_v[p, k, pl.ds(g * 16, 16)] = rows_v[p, k, pl.ds(g * 16, 16)] * b1
            return 0
        lax.fori_loop(0, PB, body, 0)

    def _sub(i, r, p, skip_wait_scatter, skip_stage2):
        q = 1 - p
        _wait_stage(i + 1, (r + 1) % 4)
        if skip_wait_scatter is None:
            _wait_scatter(q, (r + 3) % 4)
        elif skip_wait_scatter is not True:
            @pl.when(skip_wait_scatter)
            def _():
                _wait_scatter(q, (r + 3) % 4)
        pltpu.async_copy(zview.at[sp_v.at[(r + 1) % 4]], rows_v.at[q], gsem[q])
        pltpu.make_async_copy(zview.at[sp_v.at[r]], rows_v.at[p], gsem[p]).wait()
        if not skip_stage2:
            _stage(i + 2, (r + 2) % 4)
        _scale(p, r)
        pltpu.async_copy(rows_v.at[p], num_sp.at[dp_v.at[r]], ssem[p], add=True)

    _stage(0, 0)
    _stage(1, 1)
    _wait_stage(0, 0)
    pltpu.async_copy(zview.at[sp_v.at[0]], rows_v.at[0], gsem[0])

    def _quad(k, _):
        i0 = k * 4
        _sub(i0, 0, 0, skip_wait_scatter=(k > 0), skip_stage2=False)
        _sub(i0 + 1, 1, 1, skip_wait_scatter=None, skip_stage2=False)
        _sub(i0 + 2, 2, 0, skip_wait_scatter=None, skip_stage2=False)
        _sub(i0 + 3, 3, 1, skip_wait_scatter=None, skip_stage2=False)
        return 0
    lax.fori_loop(0, NCH // 4, _quad, 0)

    _wait_stage(NCH - 1, 1)
    _wait_scatter(1, 3)
    pltpu.async_copy(zview.at[sp_v.at[1]], rows_v.at[1], gsem[1])
    pltpu.make_async_copy(zview.at[sp_v.at[0]], rows_v.at[0], gsem[0]).wait()
    _scale(0, 0)
    pltpu.async_copy(rows_v.at[0], num_sp.at[dp_v.at[0]], ssem[0], add=True)
    _wait_scatter(0, 0)
    pltpu.make_async_copy(zview.at[sp_v.at[1]], rows_v.at[1], gsem[1]).wait()
    _scale(1, 1)
    pltpu.async_copy(rows_v.at[1], num_sp.at[dp_v.at[1]], ssem[1], add=True)
    _wait_scatter(1, 1)

    plsc.subcore_barrier()
    for j in range(8):
        pltpu.sync_copy(num_sp.at[pl.ds(s * NROW + j * PB, PB)],
                        num_hbm.at[c].at[pl.ds(s * NROW + j * PB, PB)])


def _edge_phase(zp, elr, mx, src, dst):
    mesh = plsc.VectorSubcoreMesh(core_axis_name="c", subcore_axis_name="s")
    f = pl.kernel(
        _sc_edge_body,
        out_type=[
            jax.ShapeDtypeStruct((2, NP, 128), jnp.float32),
            jax.ShapeDtypeStruct((2, 16, 2, DR, 128), jnp.float32),
            jax.ShapeDtypeStruct((2 * 2 * E,), jnp.float32),
        ],
        mesh=mesh,
        compiler_params=pltpu.CompilerParams(needs_layout_passes=False),
        scratch_types=[
            pltpu.VMEM((NP,), jnp.float32),
            pltpu.VMEM((NP,), jnp.float32),
            pltpu.VMEM((CB,), jnp.int32),
            pltpu.VMEM((CB,), jnp.int32),
            pltpu.VMEM((CB,), jnp.int32),
            pltpu.VMEM((CB,), jnp.int32),
            pltpu.VMEM((CB,), jnp.float32),
            pltpu.VMEM((CB,), jnp.float32),
            pltpu.VMEM((4, PB), jnp.int32),
            pltpu.VMEM((4, PB), jnp.int32),
            pltpu.VMEM((4, PB), jnp.float32),
            pltpu.VMEM((4, PB), jnp.float32),
            pltpu.VMEM((2, PB, 128), jnp.float32),
            pltpu.VMEM((16,), jnp.float32),
            pltpu.VMEM_SHARED((NP, 128), jnp.float32),
            pltpu.SemaphoreType.DMA,
            pltpu.SemaphoreType.DMA,
            pltpu.SemaphoreType.DMA,
            pltpu.SemaphoreType.DMA,
            pltpu.SemaphoreType.DMA,
            pltpu.SemaphoreType.DMA,
            pltpu.SemaphoreType.DMA,
            pltpu.SemaphoreType.DMA,
        ],
    )
    num, den, _ee = f(zp, elr.reshape(8 * NP), mx.reshape(1024), src, dst)
    return num, den.reshape(2, 16, 2, NP)


def _c_body(num_ref, den_ref, h_ref, hn_ref):
    num = jnp.concatenate([num_ref[0], num_ref[1]], axis=1)
    dp = den_ref[...].reshape(2, 16, 2, BN).sum(axis=1)
    den = dp.reshape(4, BN).T
    dd = jnp.broadcast_to(den[:, :, None], (BN, NHEAD, HD)).reshape(BN, H)
    hn_ref[...] = jax.nn.relu(num / (dd + 1e-9) + h_ref[...])


def _c(num, den, h):
    return pl.pallas_call(
        _c_body,
        grid=(GN,),
        in_specs=[
            pl.BlockSpec((2, BN, 128), lambda i: (0, i, 0)),
            pl.BlockSpec((64, BN), lambda i: (0, i)),
            pl.BlockSpec((BN, H), lambda i: (i, 0)),
        ],
        out_specs=pl.BlockSpec((BN, H), lambda i: (i, 0)),
        out_shape=jax.ShapeDtypeStruct((NP, H), jnp.float32),
    )(num, den.reshape(64, NP), h)


def _d_body(h_ref, w_ref, b_ref, xn_ref, loss_ref):
    i = pl.program_id(0)
    h = h_ref[...]
    out = jnp.dot(h, w_ref[...], preferred_element_type=jnp.float32) + b_ref[0, :][None, :]
    xn = xn_ref[...]
    no = jnp.sqrt((out * out).sum(axis=1, keepdims=True))
    nx = jnp.sqrt((xn * xn).sum(axis=1, keepdims=True))
    c = ((out / (no + 1e-12)) * (xn / (nx + 1e-12))).sum(axis=1)
    rid = i * BN + lax.broadcasted_iota(jnp.int32, (BN,), 0)
    part = jnp.where(rid < N, (1.0 - c) ** 2, 0.0).sum()

    @pl.when(i == 0)
    def _():
        loss_ref[0, 0] = 0.0

    loss_ref[0, 0] += part


def _d(h, W_out, b_out8, xn):
    return pl.pallas_call(
        _d_body,
        grid=(GN,),
        in_specs=[
            pl.BlockSpec((BN, H), lambda i: (i, 0)),
            pl.BlockSpec((H, D), lambda i: (0, 0)),
            pl.BlockSpec((8, D), lambda i: (0, 0)),
            pl.BlockSpec((BN, D), lambda i: (i, 0)),
        ],
        out_specs=pl.BlockSpec(memory_space=pltpu.SMEM),
        out_shape=jax.ShapeDtypeStruct((1, 1), jnp.float32),
    )(h, W_out, b_out8, xn)


def _head_mat(a):
    col = jnp.arange(128)[None, :]
    row_head = (jnp.arange(H) // HD)[:, None]
    return jnp.where(col == row_head, a.reshape(H)[:, None], 0.0).astype(jnp.float32)


def kernel(x, edge_index, t, noise, W_in, b_in, gat_W, gat_al, gat_ar, W_out, b_out, time_emb):
    src = edge_index[0]
    dst = edge_index[1]
    pad = NP - N
    x_pad = jnp.pad(x, ((0, pad), (0, 0)))
    nz_pad = jnp.pad(noise, ((0, pad), (0, 0)))
    t_pad = jnp.pad(t, (0, pad)).reshape(80, 128)
    te_pad = jnp.pad(time_emb, ((0, TP - T), (0, 0)))
    b_in8 = jnp.broadcast_to(b_in[None, :], (8, H))
    b_out8 = jnp.broadcast_to(b_out[None, :], (8, D))

    xn, stats = _a1(x_pad)
    h = _a2(xn, stats, nz_pad, t_pad, te_pad, W_in, b_in8)

    for l in range(L):
        zp, elr, mx = _b(h, gat_W[l], _head_mat(gat_al[l]), _head_mat(gat_ar[l]))
        num, den = _edge_phase(zp, elr, mx, src, dst)
        h = _c(num, den, h)

    loss = _d(h, W_out, b_out8, xn)
    return loss[0, 0] / N


# speedup vs baseline: 89.5397x; 1.1376x over previous
import functools
import jax
import jax.numpy as jnp
from jax import lax
from jax.experimental import pallas as pl
from jax.experimental.pallas import tpu as pltpu
from jax.experimental.pallas import tpu_sc as plsc

N = 10000
D = 128
E = 320000
H = 256
L = 2
NHEAD = 4
HD = 64
T = 1000

NP = 10240
TP = 1024
BN = 1024
GN = NP // BN


def _rln(v, eps=1e-5):
    m = v.mean(axis=-1, keepdims=True)
    var = ((v - m) ** 2).mean(axis=-1, keepdims=True)
    return (v - m) * lax.rsqrt(var + eps)


def _a1_body(x_ref, xn_ref, stats_ref):
    i = pl.program_id(0)
    xn = _rln(x_ref[...])
    xn_ref[...] = xn

    @pl.when(i == 0)
    def _():
        stats_ref[...] = jnp.zeros_like(stats_ref)

    s = xn.sum(axis=0)
    ss = (xn * xn).sum(axis=0)
    blk = jnp.concatenate([s[None, :], ss[None, :]], axis=0)
    stats_ref[0:2, :] += blk


def _a1(x_pad):
    return pl.pallas_call(
        _a1_body,
        grid=(GN,),
        in_specs=[pl.BlockSpec((BN, D), lambda i: (i, 0))],
        out_specs=[
            pl.BlockSpec((BN, D), lambda i: (i, 0)),
            pl.BlockSpec((8, D), lambda i: (0, 0)),
        ],
        out_shape=[
            jax.ShapeDtypeStruct((NP, D), jnp.float32),
            jax.ShapeDtypeStruct((8, D), jnp.float32),
        ],
    )(x_pad)


def _a2_body(xn_ref, stats_ref, nz_ref, t_ref, te_ref, w_ref, b_ref, h0_ref):
    xn = xn_ref[...]
    s = stats_ref[0, :]
    ss = stats_ref[1, :]
    miu = s / N
    var = (ss - N * miu * miu) / (N - 1)
    std = jnp.sqrt(var)

    nz = _rln(nz_ref[...])
    nz = nz * std[None, :] + miu[None, :]
    nz = jnp.sign(xn) * jnp.abs(nz)

    kk = lax.broadcasted_iota(jnp.int32, (TP, 1), 0)
    r = kk.astype(jnp.float32)
    betas = 1e-4 + r * ((0.02 - 1e-4) / (T - 1))
    alphas = jnp.where(kk < T, 1.0 - betas, 1.0)
    la = jnp.log(alphas)
    ir = lax.broadcasted_iota(jnp.int32, (TP, TP), 0)
    ic = lax.broadcasted_iota(jnp.int32, (TP, TP), 1)
    tri = (ir >= ic).astype(jnp.float32)
    cum = jnp.dot(tri, la, preferred_element_type=jnp.float32)
    ab = jnp.exp(cum)
    sa = jnp.sqrt(ab)
    sb = jnp.sqrt(jnp.maximum(1.0 - ab, 0.0))
    sasb = jnp.concatenate([sa, sb], axis=1)

    t3 = t_ref[...]
    oh = (lax.broadcasted_iota(jnp.int32, (8, 128, TP), 2)
          == t3[:, :, None]).astype(jnp.float32).reshape(BN, TP)
    g = jnp.dot(oh, sasb, preferred_element_type=jnp.float32)
    te = jnp.dot(oh, te_ref[...], preferred_element_type=jnp.float32)

    x_t = g[:, 0:1] * xn + g[:, 1:2] * nz
    h0 = jax.nn.relu(jnp.dot(x_t, w_ref[...], preferred_element_type=jnp.float32)
                     + b_ref[0, :][None, :]) + te
    h0_ref[...] = h0


def _a2(xn, stats, nz_pad, t_pad, te_pad, W_in, b_in8):
    return pl.pallas_call(
        _a2_body,
        grid=(GN,),
        in_specs=[
            pl.BlockSpec((BN, D), lambda i: (i, 0)),
            pl.BlockSpec((8, D), lambda i: (0, 0)),
            pl.BlockSpec((BN, D), lambda i: (i, 0)),
            pl.BlockSpec((8, 128), lambda i: (i, 0)),
            pl.BlockSpec((TP, H), lambda i: (0, 0)),
            pl.BlockSpec((D, H), lambda i: (0, 0)),
            pl.BlockSpec((8, H), lambda i: (0, 0)),
        ],
        out_specs=pl.BlockSpec((BN, H), lambda i: (i, 0)),
        out_shape=jax.ShapeDtypeStruct((NP, H), jnp.float32),
    )(xn, stats, nz_pad, t_pad, te_pad, W_in, b_in8)


def _b_body(h_ref, w_ref, alm_ref, arm_ref, zp_ref, elr_ref, mx_ref):
    i = pl.program_id(0)
    h = h_ref[...]
    z = jnp.dot(h, w_ref[...], preferred_element_type=jnp.float32)
    zp_ref[0, :, :] = z[:, 0:128]
    zp_ref[1, :, :] = z[:, 128:256]
    el = jnp.dot(z, alm_ref[...], preferred_element_type=jnp.float32)
    er = jnp.dot(z, arm_ref[...], preferred_element_type=jnp.float32)
    el_t = el.T
    er_t = er.T
    elr_ref[...] = jnp.concatenate([el_t[0:4, :], er_t[0:4, :]], axis=0)

    @pl.when(i == 0)
    def _():
        mx_ref[...] = jnp.full_like(mx_ref, -1e30)

    blkmax = jnp.max(el_t[0:8, :], axis=1, keepdims=True)
    mx_ref[...] = jnp.maximum(mx_ref[...], blkmax + jnp.zeros((8, 128), jnp.float32))


def _b(h, W, al_mat, ar_mat):
    return pl.pallas_call(
        _b_body,
        grid=(GN,),
        in_specs=[
            pl.BlockSpec((BN, H), lambda i: (i, 0)),
            pl.BlockSpec((H, H), lambda i: (0, 0)),
            pl.BlockSpec((H, 128), lambda i: (0, 0)),
            pl.BlockSpec((H, 128), lambda i: (0, 0)),
        ],
        out_specs=[
            pl.BlockSpec((2, BN, 128), lambda i: (0, i, 0)),
            pl.BlockSpec((8, BN), lambda i: (0, i)),
            pl.BlockSpec((8, 128), lambda i: (0, 0)),
        ],
        out_shape=[
            jax.ShapeDtypeStruct((2, NP, 128), jnp.float32),
            jax.ShapeDtypeStruct((8, NP), jnp.float32),
            jax.ShapeDtypeStruct((8, 128), jnp.float32),
        ],
    )(h, W, al_mat, ar_mat)


EC = E // 16
CB = 400
PB = 80
NROW = NP // 16
DR = NP // 128


NCH = EC // PB


def _sc_edge_body(zp_hbm, elr_hbm, mx_hbm, src_hbm, dst_hbm,
                  num_hbm, den_hbm, ee_hbm,
                  el_v, er_v, s1a_v, s1b_v, d1a_v, d1b_v, eea_v, eeb_v,
                  sp_v, dp_v, e0_v, e1_v, rows_v, mx_v, num_sp,
                  st0, st1, st2, st3, g0, g1, ss0, ss1):
    c = lax.axis_index("c")
    s = lax.axis_index("s")
    ebase = s * EC
    stsem = [st0, st1, st2, st3]
    gsem = [g0, g1]
    ssem = [ss0, ss1]
    s1l = [s1a_v, s1b_v]
    d1l = [d1a_v, d1b_v]
    eel = [eea_v, eeb_v]

    def _z1(i, _):
        rows_v[0, i // 8, pl.ds((i % 8) * 16, 16)] = jnp.zeros((16,), jnp.float32)
        return 0
    lax.fori_loop(0, PB * 8, _z1, 0)
    for j in range(8):
        pltpu.sync_copy(rows_v.at[0], num_sp.at[pl.ds(s * NROW + j * PB, PB)])

    zero16 = jnp.zeros((16,), jnp.int32)
    NC1 = EC // CB

    for hh in range(2):
        h = 2 * c + hh
        pltpu.sync_copy(elr_hbm.at[pl.ds(h * NP, NP)], el_v)
        pltpu.sync_copy(elr_hbm.at[pl.ds((4 + h) * NP, NP)], er_v)
        pltpu.sync_copy(mx_hbm.at[pl.ds(h * 128, 16)], mx_v)

        def _zd(i, _):
            rows_v[0, i // 8, pl.ds((i % 8) * 16, 16)] = jnp.zeros((16,), jnp.float32)
            return 0
        lax.fori_loop(0, DR * 8, _zd, 0)

        mxb = mx_v[...]

        def _stage1(i, u):
            pltpu.async_copy(src_hbm.at[pl.ds(ebase + i * CB, CB)], s1l[u], stsem[u])
            pltpu.async_copy(dst_hbm.at[pl.ds(ebase + i * CB, CB)], d1l[u], stsem[u])

        def _wait_stage1(i, u):
            pltpu.make_async_copy(src_hbm.at[pl.ds(ebase + i * CB, CB)], s1l[u], stsem[u]).wait()
            pltpu.make_async_copy(dst_hbm.at[pl.ds(ebase + i * CB, CB)], d1l[u], stsem[u]).wait()

        def _ee_off(i):
            return (2 * c + hh) * E + ebase + i * CB

        def _sub1(i, u, wait_wb, last):
            _wait_stage1(i, u)
            if not last:
                @pl.when(i + 1 < NC1)
                def _():
                    _stage1(i + 1, 1 - u)
            if wait_wb is not None:
                @pl.when(wait_wb)
                def _():
                    pltpu.make_async_copy(eel[u], ee_hbm.at[pl.ds(_ee_off(i - 2), CB)],
                                          stsem[2 + u]).wait()

            def _grp(j, _):
                s16 = s1l[u][pl.ds(j * 16, 16)]
                d16 = d1l[u][pl.ds(j * 16, 16)]
                gel = plsc.load_gather(el_v, [s16])
                ger = plsc.load_gather(er_v, [d16])
                a = gel + ger
                e = jnp.where(a > 0, a, 0.2 * a)
                u2 = mxb + ger
                u2 = jnp.where(u2 > 0, u2, 0.2 * u2)
                ee = jnp.exp(e - u2)
                r16 = lax.shift_right_logical(d16, 7)
                c16 = lax.bitwise_and(d16, 127)
                plsc.addupdate_scatter(rows_v, [zero16, r16, c16], ee)
                eel[u][pl.ds(j * 16, 16)] = ee
                return 0
            lax.fori_loop(0, CB // 16, _grp, 0)
            pltpu.async_copy(eel[u], ee_hbm.at[pl.ds(_ee_off(i), CB)], stsem[2 + u])

        _stage1(0, 0)

        def _pair1(k, _):
            i0 = 2 * k
            _sub1(i0, 0, (k > 0), False)
            _sub1(i0 + 1, 1, (k > 0), False)
            return 0
        lax.fori_loop(0, NC1 // 2 - 1, _pair1, 0)
        _sub1(NC1 - 2, 0, True, False)
        _sub1(NC1 - 1, 1, True, True)
        for u in range(2):
            pltpu.make_async_copy(eel[u], ee_hbm.at[pl.ds(_ee_off(NC1 - 2 + u), CB)],
                                  stsem[2 + u]).wait()
        pltpu.sync_copy(rows_v.at[0], den_hbm.at[c, s, hh])

    plsc.subcore_barrier()
    zview = zp_hbm.at[c]

    def _stage(i, r):
        off = ebase + i * PB
        pltpu.async_copy(src_hbm.at[pl.ds(off, PB)], sp_v.at[r], stsem[r])
        pltpu.async_copy(dst_hbm.at[pl.ds(off, PB)], dp_v.at[r], stsem[r])
        pltpu.async_copy(ee_hbm.at[pl.ds(2 * c * E + off, PB)], e0_v.at[r], stsem[r])
        pltpu.async_copy(ee_hbm.at[pl.ds((2 * c + 1) * E + off, PB)], e1_v.at[r], stsem[r])

    def _wait_stage(i, r):
        off = ebase + i * PB
        pltpu.make_async_copy(src_hbm.at[pl.ds(off, PB)], sp_v.at[r], stsem[r]).wait()
        pltpu.make_async_copy(dst_hbm.at[pl.ds(off, PB)], dp_v.at[r], stsem[r]).wait()
        pltpu.make_async_copy(ee_hbm.at[pl.ds(2 * c * E + off, PB)], e0_v.at[r], stsem[r]).wait()
        pltpu.make_async_copy(ee_hbm.at[pl.ds((2 * c + 1) * E + off, PB)], e1_v.at[r], stsem[r]).wait()

    def _wait_scatter(p, r):
        pltpu.make_async_copy(rows_v.at[p], num_sp.at[dp_v.at[r]], ssem[p]).wait()

    def _scale(p, r):
        r16 = jnp.full((16,), r, jnp.int32)

        @plsc.parallel_loop(0, PB, 1, unroll=4)
        def _(k):
            i16 = jnp.broadcast_to(k, (16,))
            b0 = plsc.load_gather(e0_v, [r16, i16])
            b1 = plsc.load_gather(e1_v, [r16, i16])
            for g in range(4):
                rows_v[p, k, pl.ds(g * 16, 16)] = rows_v[p, k, pl.ds(g * 16, 16)] * b0
            for g in range(4, 8):
                rows_v[p, k, pl.ds(g * 16, 16)] = rows_v[p, k, pl.ds(g * 16, 16)] * b1

    def _sub(i, r, p, skip_wait_scatter, skip_stage2):
        q = 1 - p
        _wait_stage(i + 1, (r + 1) % 4)
        if skip_wait_scatter is None:
            _wait_scatter(q, (r + 3) % 4)
        elif skip_wait_scatter is not True:
            @pl.when(skip_wait_scatter)
            def _():
                _wait_scatter(q, (r + 3) % 4)
        pltpu.async_copy(zview.at[sp_v.at[(r + 1) % 4]], rows_v.at[q], gsem[q])
        pltpu.make_async_copy(zview.at[sp_v.at[r]], rows_v.at[p], gsem[p]).wait()
        if not skip_stage2:
            _stage(i + 2, (r + 2) % 4)
        _scale(p, r)
        pltpu.async_copy(rows_v.at[p], num_sp.at[dp_v.at[r]], ssem[p], add=True)

    _stage(0, 0)
    _stage(1, 1)
    _wait_stage(0, 0)
    pltpu.async_copy(zview.at[sp_v.at[0]], rows_v.at[0], gsem[0])

    def _quad(k, _):
        i0 = k * 4
        _sub(i0, 0, 0, skip_wait_scatter=(k > 0), skip_stage2=False)
        _sub(i0 + 1, 1, 1, skip_wait_scatter=None, skip_stage2=False)
        _sub(i0 + 2, 2, 0, skip_wait_scatter=None, skip_stage2=False)
        _sub(i0 + 3, 3, 1, skip_wait_scatter=None, skip_stage2=False)
        return 0
    lax.fori_loop(0, NCH // 4, _quad, 0)

    _wait_stage(NCH - 1, 1)
    _wait_scatter(1, 3)
    pltpu.async_copy(zview.at[sp_v.at[1]], rows_v.at[1], gsem[1])
    pltpu.make_async_copy(zview.at[sp_v.at[0]], rows_v.at[0], gsem[0]).wait()
    _scale(0, 0)
    pltpu.async_copy(rows_v.at[0], num_sp.at[dp_v.at[0]], ssem[0], add=True)
    _wait_scatter(0, 0)
    pltpu.make_async_copy(zview.at[sp_v.at[1]], rows_v.at[1], gsem[1]).wait()
    _scale(1, 1)
    pltpu.async_copy(rows_v.at[1], num_sp.at[dp_v.at[1]], ssem[1], add=True)
    _wait_scatter(1, 1)

    plsc.subcore_barrier()
    for j in range(8):
        pltpu.sync_copy(num_sp.at[pl.ds(s * NROW + j * PB, PB)],
                        num_hbm.at[c].at[pl.ds(s * NROW + j * PB, PB)])


def _edge_phase(zp, elr, mx, src, dst):
    mesh = plsc.VectorSubcoreMesh(core_axis_name="c", subcore_axis_name="s")
    f = pl.kernel(
        _sc_edge_body,
        out_type=[
            jax.ShapeDtypeStruct((2, NP, 128), jnp.float32),
            jax.ShapeDtypeStruct((2, 16, 2, DR, 128), jnp.float32),
            jax.ShapeDtypeStruct((2 * 2 * E,), jnp.float32),
        ],
        mesh=mesh,
        compiler_params=pltpu.CompilerParams(needs_layout_passes=False),
        scratch_types=[
            pltpu.VMEM((NP,), jnp.float32),
            pltpu.VMEM((NP,), jnp.float32),
            pltpu.VMEM((CB,), jnp.int32),
            pltpu.VMEM((CB,), jnp.int32),
            pltpu.VMEM((CB,), jnp.int32),
            pltpu.VMEM((CB,), jnp.int32),
            pltpu.VMEM((CB,), jnp.float32),
            pltpu.VMEM((CB,), jnp.float32),
            pltpu.VMEM((4, PB), jnp.int32),
            pltpu.VMEM((4, PB), jnp.int32),
            pltpu.VMEM((4, PB), jnp.float32),
            pltpu.VMEM((4, PB), jnp.float32),
            pltpu.VMEM((2, PB, 128), jnp.float32),
            pltpu.VMEM((16,), jnp.float32),
            pltpu.VMEM_SHARED((NP, 128), jnp.float32),
            pltpu.SemaphoreType.DMA,
            pltpu.SemaphoreType.DMA,
            pltpu.SemaphoreType.DMA,
            pltpu.SemaphoreType.DMA,
            pltpu.SemaphoreType.DMA,
            pltpu.SemaphoreType.DMA,
            pltpu.SemaphoreType.DMA,
            pltpu.SemaphoreType.DMA,
        ],
    )
    num, den, _ee = f(zp, elr.reshape(8 * NP), mx.reshape(1024), src, dst)
    return num, den.reshape(2, 16, 2, NP)


def _c_body(num_ref, den_ref, h_ref, hn_ref):
    num = jnp.concatenate([num_ref[0], num_ref[1]], axis=1)
    dp = den_ref[...].reshape(2, 16, 2, BN).sum(axis=1)
    den = dp.reshape(4, BN).T
    dd = jnp.broadcast_to(den[:, :, None], (BN, NHEAD, HD)).reshape(BN, H)
    hn_ref[...] = jax.nn.relu(num / (dd + 1e-9) + h_ref[...])


def _c(num, den, h):
    return pl.pallas_call(
        _c_body,
        grid=(GN,),
        in_specs=[
            pl.BlockSpec((2, BN, 128), lambda i: (0, i, 0)),
            pl.BlockSpec((64, BN), lambda i: (0, i)),
            pl.BlockSpec((BN, H), lambda i: (i, 0)),
        ],
        out_specs=pl.BlockSpec((BN, H), lambda i: (i, 0)),
        out_shape=jax.ShapeDtypeStruct((NP, H), jnp.float32),
    )(num, den.reshape(64, NP), h)


def _d_body(h_ref, w_ref, b_ref, xn_ref, loss_ref):
    i = pl.program_id(0)
    h = h_ref[...]
    out = jnp.dot(h, w_ref[...], preferred_element_type=jnp.float32) + b_ref[0, :][None, :]
    xn = xn_ref[...]
    no = jnp.sqrt((out * out).sum(axis=1, keepdims=True))
    nx = jnp.sqrt((xn * xn).sum(axis=1, keepdims=True))
    c = ((out / (no + 1e-12)) * (xn / (nx + 1e-12))).sum(axis=1)
    rid = i * BN + lax.broadcasted_iota(jnp.int32, (BN,), 0)
    part = jnp.where(rid < N, (1.0 - c) ** 2, 0.0).sum()

    @pl.when(i == 0)
    def _():
        loss_ref[0, 0] = 0.0

    loss_ref[0, 0] += part


def _d(h, W_out, b_out8, xn):
    return pl.pallas_call(
        _d_body,
        grid=(GN,),
        in_specs=[
            pl.BlockSpec((BN, H), lambda i: (i, 0)),
            pl.BlockSpec((H, D), lambda i: (0, 0)),
            pl.BlockSpec((8, D), lambda i: (0, 0)),
            pl.BlockSpec((BN, D), lambda i: (i, 0)),
        ],
        out_specs=pl.BlockSpec(memory_space=pltpu.SMEM),
        out_shape=jax.ShapeDtypeStruct((1, 1), jnp.float32),
    )(h, W_out, b_out8, xn)


def _head_mat(a):
    col = jnp.arange(128)[None, :]
    row_head = (jnp.arange(H) // HD)[:, None]
    return jnp.where(col == row_head, a.reshape(H)[:, None], 0.0).astype(jnp.float32)


def kernel(x, edge_index, t, noise, W_in, b_in, gat_W, gat_al, gat_ar, W_out, b_out, time_emb):
    src = edge_index[0]
    dst = edge_index[1]
    pad = NP - N
    x_pad = jnp.pad(x, ((0, pad), (0, 0)))
    nz_pad = jnp.pad(noise, ((0, pad), (0, 0)))
    t_pad = jnp.pad(t, (0, pad)).reshape(80, 128)
    te_pad = jnp.pad(time_emb, ((0, TP - T), (0, 0)))
    b_in8 = jnp.broadcast_to(b_in[None, :], (8, H))
    b_out8 = jnp.broadcast_to(b_out[None, :], (8, D))

    xn, stats = _a1(x_pad)
    h = _a2(xn, stats, nz_pad, t_pad, te_pad, W_in, b_in8)

    for l in range(L):
        zp, elr, mx = _b(h, gat_W[l], _head_mat(gat_al[l]), _head_mat(gat_ar[l]))
        num, den = _edge_phase(zp, elr, mx, src, dst)
        h = _c(num, den, h)

    loss = _d(h, W_out, b_out8, xn)
    return loss[0, 0] / N


# speedup vs baseline: 89.7720x; 1.0026x over previous
import functools
import jax
import jax.numpy as jnp
from jax import lax
from jax.experimental import pallas as pl
from jax.experimental.pallas import tpu as pltpu
from jax.experimental.pallas import tpu_sc as plsc

N = 10000
D = 128
E = 320000
H = 256
L = 2
NHEAD = 4
HD = 64
T = 1000

NP = 10240
TP = 1024
BN = 1024
GN = NP // BN


def _rln(v, eps=1e-5):
    m = v.mean(axis=-1, keepdims=True)
    var = ((v - m) ** 2).mean(axis=-1, keepdims=True)
    return (v - m) * lax.rsqrt(var + eps)


def _a1_body(x_ref, xn_ref, stats_ref):
    i = pl.program_id(0)
    xn = _rln(x_ref[...])
    xn_ref[...] = xn

    @pl.when(i == 0)
    def _():
        stats_ref[...] = jnp.zeros_like(stats_ref)

    s = xn.sum(axis=0)
    ss = (xn * xn).sum(axis=0)
    blk = jnp.concatenate([s[None, :], ss[None, :]], axis=0)
    stats_ref[0:2, :] += blk


def _a1(x_pad):
    return pl.pallas_call(
        _a1_body,
        grid=(GN,),
        in_specs=[pl.BlockSpec((BN, D), lambda i: (i, 0))],
        out_specs=[
            pl.BlockSpec((BN, D), lambda i: (i, 0)),
            pl.BlockSpec((8, D), lambda i: (0, 0)),
        ],
        out_shape=[
            jax.ShapeDtypeStruct((NP, D), jnp.float32),
            jax.ShapeDtypeStruct((8, D), jnp.float32),
        ],
    )(x_pad)


def _a2_body(xn_ref, stats_ref, nz_ref, t_ref, te_ref, w_ref, b_ref, h0_ref):
    xn = xn_ref[...]
    s = stats_ref[0, :]
    ss = stats_ref[1, :]
    miu = s / N
    var = (ss - N * miu * miu) / (N - 1)
    std = jnp.sqrt(var)

    nz = _rln(nz_ref[...])
    nz = nz * std[None, :] + miu[None, :]
    nz = jnp.sign(xn) * jnp.abs(nz)

    kk = lax.broadcasted_iota(jnp.int32, (TP, 1), 0)
    r = kk.astype(jnp.float32)
    betas = 1e-4 + r * ((0.02 - 1e-4) / (T - 1))
    alphas = jnp.where(kk < T, 1.0 - betas, 1.0)
    la = jnp.log(alphas)
    ir = lax.broadcasted_iota(jnp.int32, (TP, TP), 0)
    ic = lax.broadcasted_iota(jnp.int32, (TP, TP), 1)
    tri = (ir >= ic).astype(jnp.float32)
    cum = jnp.dot(tri, la, preferred_element_type=jnp.float32)
    ab = jnp.exp(cum)
    sa = jnp.sqrt(ab)
    sb = jnp.sqrt(jnp.maximum(1.0 - ab, 0.0))
    sasb = jnp.concatenate([sa, sb], axis=1)

    t3 = t_ref[...]
    oh = (lax.broadcasted_iota(jnp.int32, (8, 128, TP), 2)
          == t3[:, :, None]).astype(jnp.float32).reshape(BN, TP)
    g = jnp.dot(oh, sasb, preferred_element_type=jnp.float32)
    te = jnp.dot(oh, te_ref[...], preferred_element_type=jnp.float32)

    x_t = g[:, 0:1] * xn + g[:, 1:2] * nz
    h0 = jax.nn.relu(jnp.dot(x_t, w_ref[...], preferred_element_type=jnp.float32)
                     + b_ref[0, :][None, :]) + te
    h0_ref[...] = h0


def _a2(xn, stats, nz_pad, t_pad, te_pad, W_in, b_in8):
    return pl.pallas_call(
        _a2_body,
        grid=(GN,),
        in_specs=[
            pl.BlockSpec((BN, D), lambda i: (i, 0)),
            pl.BlockSpec((8, D), lambda i: (0, 0)),
            pl.BlockSpec((BN, D), lambda i: (i, 0)),
            pl.BlockSpec((8, 128), lambda i: (i, 0)),
            pl.BlockSpec((TP, H), lambda i: (0, 0)),
            pl.BlockSpec((D, H), lambda i: (0, 0)),
            pl.BlockSpec((8, H), lambda i: (0, 0)),
        ],
        out_specs=pl.BlockSpec((BN, H), lambda i: (i, 0)),
        out_shape=jax.ShapeDtypeStruct((NP, H), jnp.float32),
    )(xn, stats, nz_pad, t_pad, te_pad, W_in, b_in8)


def _b_body(h_ref, w_ref, alm_ref, arm_ref, zp_ref, elr_ref, mx_ref):
    i = pl.program_id(0)
    h = h_ref[...]
    z = jnp.dot(h, w_ref[...], preferred_element_type=jnp.float32)
    zp_ref[0, :, :] = z[:, 0:128]
    zp_ref[1, :, :] = z[:, 128:256]
    el = jnp.dot(z, alm_ref[...], preferred_element_type=jnp.float32)
    er = jnp.dot(z, arm_ref[...], preferred_element_type=jnp.float32)
    el_t = el.T
    er_t = er.T
    elr_ref[...] = jnp.concatenate([el_t[0:4, :], er_t[0:4, :]], axis=0)

    @pl.when(i == 0)
    def _():
        mx_ref[...] = jnp.full_like(mx_ref, -1e30)

    blkmax = jnp.max(el_t[0:8, :], axis=1, keepdims=True)
    mx_ref[...] = jnp.maximum(mx_ref[...], blkmax + jnp.zeros((8, 128), jnp.float32))


def _b(h, W, al_mat, ar_mat):
    return pl.pallas_call(
        _b_body,
        grid=(GN,),
        in_specs=[
            pl.BlockSpec((BN, H), lambda i: (i, 0)),
            pl.BlockSpec((H, H), lambda i: (0, 0)),
            pl.BlockSpec((H, 128), lambda i: (0, 0)),
            pl.BlockSpec((H, 128), lambda i: (0, 0)),
        ],
        out_specs=[
            pl.BlockSpec((2, BN, 128), lambda i: (0, i, 0)),
            pl.BlockSpec((8, BN), lambda i: (0, i)),
            pl.BlockSpec((8, 128), lambda i: (0, 0)),
        ],
        out_shape=[
            jax.ShapeDtypeStruct((2, NP, 128), jnp.float32),
            jax.ShapeDtypeStruct((8, NP), jnp.float32),
            jax.ShapeDtypeStruct((8, 128), jnp.float32),
        ],
    )(h, W, al_mat, ar_mat)


EC = E // 16
CB = 400
PB = 80
NROW = NP // 16
DR = NP // 128


NCH = EC // PB


def _sc_edge_body(zp_hbm, elr_hbm, mx_hbm, src_hbm, dst_hbm,
                  num_hbm, den_hbm, ee_hbm,
                  el_v, er_v, s1a_v, s1b_v, d1a_v, d1b_v, eea_v, eeb_v,
                  sp_v, dp_v, e0_v, e1_v, rows_v, mx_v, num_sp,
                  st0, st1, st2, st3, g0, g1, ss0, ss1):
    c = lax.axis_index("c")
    s = lax.axis_index("s")
    ebase = s * EC
    stsem = [st0, st1, st2, st3]
    gsem = [g0, g1]
    ssem = [ss0, ss1]
    s1l = [s1a_v, s1b_v]
    d1l = [d1a_v, d1b_v]
    eel = [eea_v, eeb_v]

    def _z1(i, _):
        rows_v[0, i // 8, pl.ds((i % 8) * 16, 16)] = jnp.zeros((16,), jnp.float32)
        return 0
    lax.fori_loop(0, PB * 8, _z1, 0)
    for j in range(8):
        pltpu.sync_copy(rows_v.at[0], num_sp.at[pl.ds(s * NROW + j * PB, PB)])

    zero16 = jnp.zeros((16,), jnp.int32)
    NC1 = EC // CB

    for hh in range(2):
        h = 2 * c + hh
        pltpu.sync_copy(elr_hbm.at[pl.ds(h * NP, NP)], el_v)
        pltpu.sync_copy(elr_hbm.at[pl.ds((4 + h) * NP, NP)], er_v)
        pltpu.sync_copy(mx_hbm.at[pl.ds(h * 128, 16)], mx_v)

        def _zd(i, _):
            rows_v[0, i // 8, pl.ds((i % 8) * 16, 16)] = jnp.zeros((16,), jnp.float32)
            return 0
        lax.fori_loop(0, DR * 8, _zd, 0)

        mxb = mx_v[...]

        def _stage1(i, u):
            pltpu.async_copy(src_hbm.at[pl.ds(ebase + i * CB, CB)], s1l[u], stsem[u])
            pltpu.async_copy(dst_hbm.at[pl.ds(ebase + i * CB, CB)], d1l[u], stsem[u])

        def _wait_stage1(i, u):
            pltpu.make_async_copy(src_hbm.at[pl.ds(ebase + i * CB, CB)], s1l[u], stsem[u]).wait()
            pltpu.make_async_copy(dst_hbm.at[pl.ds(ebase + i * CB, CB)], d1l[u], stsem[u]).wait()

        def _ee_off(i):
            return (2 * c + hh) * E + ebase + i * CB

        def _sub1(i, u, wait_wb, last):
            _wait_stage1(i, u)
            if not last:
                @pl.when(i + 1 < NC1)
                def _():
                    _stage1(i + 1, 1 - u)
            if wait_wb is not None:
                @pl.when(wait_wb)
                def _():
                    pltpu.make_async_copy(eel[u], ee_hbm.at[pl.ds(_ee_off(i - 2), CB)],
                                          stsem[2 + u]).wait()

            @plsc.parallel_loop(0, CB // 16, 1, unroll=4)
            def _grp(j):
                s16 = s1l[u][pl.ds(j * 16, 16)]
                d16 = d1l[u][pl.ds(j * 16, 16)]
                gel = plsc.load_gather(el_v, [s16])
                ger = plsc.load_gather(er_v, [d16])
                a = gel + ger
                e = jnp.where(a > 0, a, 0.2 * a)
                u2 = mxb + ger
                u2 = jnp.where(u2 > 0, u2, 0.2 * u2)
                ee = jnp.exp(e - u2)
                r16 = lax.shift_right_logical(d16, 7)
                c16 = lax.bitwise_and(d16, 127)
                plsc.addupdate_scatter(rows_v, [zero16, r16, c16], ee)
                eel[u][pl.ds(j * 16, 16)] = ee
            pltpu.async_copy(eel[u], ee_hbm.at[pl.ds(_ee_off(i), CB)], stsem[2 + u])

        _stage1(0, 0)

        def _pair1(k, _):
            i0 = 2 * k
            _sub1(i0, 0, (k > 0), False)
            _sub1(i0 + 1, 1, (k > 0), False)
            return 0
        lax.fori_loop(0, NC1 // 2 - 1, _pair1, 0)
        _sub1(NC1 - 2, 0, True, False)
        _sub1(NC1 - 1, 1, True, True)
        for u in range(2):
            pltpu.make_async_copy(eel[u], ee_hbm.at[pl.ds(_ee_off(NC1 - 2 + u), CB)],
                                  stsem[2 + u]).wait()
        pltpu.sync_copy(rows_v.at[0], den_hbm.at[c, s, hh])

    plsc.subcore_barrier()
    zview = zp_hbm.at[c]

    def _stage(i, r):
        off = ebase + i * PB
        pltpu.async_copy(src_hbm.at[pl.ds(off, PB)], sp_v.at[r], stsem[r])
        pltpu.async_copy(dst_hbm.at[pl.ds(off, PB)], dp_v.at[r], stsem[r])
        pltpu.async_copy(ee_hbm.at[pl.ds(2 * c * E + off, PB)], e0_v.at[r], stsem[r])
        pltpu.async_copy(ee_hbm.at[pl.ds((2 * c + 1) * E + off, PB)], e1_v.at[r], stsem[r])

    def _wait_stage(i, r):
        off = ebase + i * PB
        pltpu.make_async_copy(src_hbm.at[pl.ds(off, PB)], sp_v.at[r], stsem[r]).wait()
        pltpu.make_async_copy(dst_hbm.at[pl.ds(off, PB)], dp_v.at[r], stsem[r]).wait()
        pltpu.make_async_copy(ee_hbm.at[pl.ds(2 * c * E + off, PB)], e0_v.at[r], stsem[r]).wait()
        pltpu.make_async_copy(ee_hbm.at[pl.ds((2 * c + 1) * E + off, PB)], e1_v.at[r], stsem[r]).wait()

    def _wait_scatter(p, r):
        pltpu.make_async_copy(rows_v.at[p], num_sp.at[dp_v.at[r]], ssem[p]).wait()

    def _scale(p, r):
        r16 = jnp.full((16,), r, jnp.int32)

        @plsc.parallel_loop(0, PB, 1, unroll=4)
        def _(k):
            i16 = jnp.broadcast_to(k, (16,))
            b0 = plsc.load_gather(e0_v, [r16, i16])
            b1 = plsc.load_gather(e1_v, [r16, i16])
            for g in range(4):
                rows_v[p, k, pl.ds(g * 16, 16)] = rows_v[p, k, pl.ds(g * 16, 16)] * b0
            for g in range(4, 8):
                rows_v[p, k, pl.ds(g * 16, 16)] = rows_v[p, k, pl.ds(g * 16, 16)] * b1

    def _sub(i, r, p, skip_wait_scatter, skip_stage2):
        q = 1 - p
        _wait_stage(i + 1, (r + 1) % 4)
        if skip_wait_scatter is None:
            _wait_scatter(q, (r + 3) % 4)
        elif skip_wait_scatter is not True:
            @pl.when(skip_wait_scatter)
            def _():
                _wait_scatter(q, (r + 3) % 4)
        pltpu.async_copy(zview.at[sp_v.at[(r + 1) % 4]], rows_v.at[q], gsem[q])
        pltpu.make_async_copy(zview.at[sp_v.at[r]], rows_v.at[p], gsem[p]).wait()
        if not skip_stage2:
            _stage(i + 2, (r + 2) % 4)
        _scale(p, r)
        pltpu.async_copy(rows_v.at[p], num_sp.at[dp_v.at[r]], ssem[p], add=True)

    _stage(0, 0)
    _stage(1, 1)
    _wait_stage(0, 0)
    pltpu.async_copy(zview.at[sp_v.at[0]], rows_v.at[0], gsem[0])

    def _quad(k, _):
        i0 = k * 4
        _sub(i0, 0, 0, skip_wait_scatter=(k > 0), skip_stage2=False)
        _sub(i0 + 1, 1, 1, skip_wait_scatter=None, skip_stage2=False)
        _sub(i0 + 2, 2, 0, skip_wait_scatter=None, skip_stage2=False)
        _sub(i0 + 3, 3, 1, skip_wait_scatter=None, skip_stage2=False)
        return 0
    lax.fori_loop(0, NCH // 4, _quad, 0)

    _wait_stage(NCH - 1, 1)
    _wait_scatter(1, 3)
    pltpu.async_copy(zview.at[sp_v.at[1]], rows_v.at[1], gsem[1])
    pltpu.make_async_copy(zview.at[sp_v.at[0]], rows_v.at[0], gsem[0]).wait()
    _scale(0, 0)
    pltpu.async_copy(rows_v.at[0], num_sp.at[dp_v.at[0]], ssem[0], add=True)
    _wait_scatter(0, 0)
    pltpu.make_async_copy(zview.at[sp_v.at[1]], rows_v.at[1], gsem[1]).wait()
    _scale(1, 1)
    pltpu.async_copy(rows_v.at[1], num_sp.at[dp_v.at[1]], ssem[1], add=True)
    _wait_scatter(1, 1)

    plsc.subcore_barrier()
    for j in range(8):
        pltpu.sync_copy(num_sp.at[pl.ds(s * NROW + j * PB, PB)],
                        num_hbm.at[c].at[pl.ds(s * NROW + j * PB, PB)])


def _edge_phase(zp, elr, mx, src, dst):
    mesh = plsc.VectorSubcoreMesh(core_axis_name="c", subcore_axis_name="s")
    f = pl.kernel(
        _sc_edge_body,
        out_type=[
            jax.ShapeDtypeStruct((2, NP, 128), jnp.float32),
            jax.ShapeDtypeStruct((2, 16, 2, DR, 128), jnp.float32),
            jax.ShapeDtypeStruct((2 * 2 * E,), jnp.float32),
        ],
        mesh=mesh,
        compiler_params=pltpu.CompilerParams(needs_layout_passes=False),
        scratch_types=[
            pltpu.VMEM((NP,), jnp.float32),
            pltpu.VMEM((NP,), jnp.float32),
            pltpu.VMEM((CB,), jnp.int32),
            pltpu.VMEM((CB,), jnp.int32),
            pltpu.VMEM((CB,), jnp.int32),
            pltpu.VMEM((CB,), jnp.int32),
            pltpu.VMEM((CB,), jnp.float32),
            pltpu.VMEM((CB,), jnp.float32),
            pltpu.VMEM((4, PB), jnp.int32),
            pltpu.VMEM((4, PB), jnp.int32),
            pltpu.VMEM((4, PB), jnp.float32),
            pltpu.VMEM((4, PB), jnp.float32),
            pltpu.VMEM((2, PB, 128), jnp.float32),
            pltpu.VMEM((16,), jnp.float32),
            pltpu.VMEM_SHARED((NP, 128), jnp.float32),
            pltpu.SemaphoreType.DMA,
            pltpu.SemaphoreType.DMA,
            pltpu.SemaphoreType.DMA,
            pltpu.SemaphoreType.DMA,
            pltpu.SemaphoreType.DMA,
            pltpu.SemaphoreType.DMA,
            pltpu.SemaphoreType.DMA,
            pltpu.SemaphoreType.DMA,
        ],
    )
    num, den, _ee = f(zp, elr.reshape(8 * NP), mx.reshape(1024), src, dst)
    return num, den.reshape(2, 16, 2, NP)


def _c_body(num_ref, den_ref, h_ref, hn_ref):
    num = jnp.concatenate([num_ref[0], num_ref[1]], axis=1)
    dp = den_ref[...].reshape(2, 16, 2, BN).sum(axis=1)
    den = dp.reshape(4, BN).T
    dd = jnp.broadcast_to(den[:, :, None], (BN, NHEAD, HD)).reshape(BN, H)
    hn_ref[...] = jax.nn.relu(num / (dd + 1e-9) + h_ref[...])


def _c(num, den, h):
    return pl.pallas_call(
        _c_body,
        grid=(GN,),
        in_specs=[
            pl.BlockSpec((2, BN, 128), lambda i: (0, i, 0)),
            pl.BlockSpec((64, BN), lambda i: (0, i)),
            pl.BlockSpec((BN, H), lambda i: (i, 0)),
        ],
        out_specs=pl.BlockSpec((BN, H), lambda i: (i, 0)),
        out_shape=jax.ShapeDtypeStruct((NP, H), jnp.float32),
    )(num, den.reshape(64, NP), h)


def _d_body(h_ref, w_ref, b_ref, xn_ref, loss_ref):
    i = pl.program_id(0)
    h = h_ref[...]
    out = jnp.dot(h, w_ref[...], preferred_element_type=jnp.float32) + b_ref[0, :][None, :]
    xn = xn_ref[...]
    no = jnp.sqrt((out * out).sum(axis=1, keepdims=True))
    nx = jnp.sqrt((xn * xn).sum(axis=1, keepdims=True))
    c = ((out / (no + 1e-12)) * (xn / (nx + 1e-12))).sum(axis=1)
    rid = i * BN + lax.broadcasted_iota(jnp.int32, (BN,), 0)
    part = jnp.where(rid < N, (1.0 - c) ** 2, 0.0).sum()

    @pl.when(i == 0)
    def _():
        loss_ref[0, 0] = 0.0

    loss_ref[0, 0] += part


def _d(h, W_out, b_out8, xn):
    return pl.pallas_call(
        _d_body,
        grid=(GN,),
        in_specs=[
            pl.BlockSpec((BN, H), lambda i: (i, 0)),
            pl.BlockSpec((H, D), lambda i: (0, 0)),
            pl.BlockSpec((8, D), lambda i: (0, 0)),
            pl.BlockSpec((BN, D), lambda i: (i, 0)),
        ],
        out_specs=pl.BlockSpec(memory_space=pltpu.SMEM),
        out_shape=jax.ShapeDtypeStruct((1, 1), jnp.float32),
    )(h, W_out, b_out8, xn)


def _head_mat(a):
    col = jnp.arange(128)[None, :]
    row_head = (jnp.arange(H) // HD)[:, None]
    return jnp.where(col == row_head, a.reshape(H)[:, None], 0.0).astype(jnp.float32)


def kernel(x, edge_index, t, noise, W_in, b_in, gat_W, gat_al, gat_ar, W_out, b_out, time_emb):
    src = edge_index[0]
    dst = edge_index[1]
    pad = NP - N
    x_pad = jnp.pad(x, ((0, pad), (0, 0)))
    nz_pad = jnp.pad(noise, ((0, pad), (0, 0)))
    t_pad = jnp.pad(t, (0, pad)).reshape(80, 128)
    te_pad = jnp.pad(time_emb, ((0, TP - T), (0, 0)))
    b_in8 = jnp.broadcast_to(b_in[None, :], (8, H))
    b_out8 = jnp.broadcast_to(b_out[None, :], (8, D))

    xn, stats = _a1(x_pad)
    h = _a2(xn, stats, nz_pad, t_pad, te_pad, W_in, b_in8)

    for l in range(L):
        zp, elr, mx = _b(h, gat_W[l], _head_mat(gat_al[l]), _head_mat(gat_ar[l]))
        num, den = _edge_phase(zp, elr, mx, src, dst)
        h = _c(num, den, h)

    loss = _d(h, W_out, b_out8, xn)
    return loss[0, 0] / N
